# Initial kernel scaffold; baseline (speedup 1.0000x reference)
import jax
import jax.numpy as jnp
from jax.experimental import pallas as pl


def kernel(x, pos_enc, params, edge_index, batch, pos_index, pos_batch):
    raise NotImplementedError("write your pallas kernel here")


# speedup vs baseline: 1.6651x; 1.6651x over previous
import functools

import jax
import jax.numpy as jnp
from jax import lax
from jax.experimental import pallas as pl
from jax.experimental.pallas import tpu as pltpu
from jax.experimental.pallas import tpu_sc as plsc

N_NODES = 10000
N_EDGES = 160000
D = 256
HALF = 128
NUM_LAYERS = 4
Z_IN = 1800
P_POS = 320000
N_GRAPHS = 128
BN_C = (1.0 + 1e-5) ** -0.5

def _sread(ref, i):
    return ref[pl.ds(i, 16)][0]


NW = 32
E_PER_W = N_EDGES // NW
EC = 200
PBUF = 128
K3_EW = N_EDGES // 16
K3_C = 80
K3_NC = K3_EW // K3_C
NODE_NC = N_NODES // K3_C


def _k1_body(zinit_hbm, pidx_hbm, pe_hbm, pb_hbm, pcnt_hbm, zraw_hbm,
             pcnt_v, pb_v, pe_v, pidx_v, zrows_v, acc_v, sem):
    c = lax.axis_index("c")
    s = lax.axis_index("s")
    w = s * 2 + c
    e0 = w * E_PER_W
    pltpu.async_copy(pcnt_hbm, pcnt_v, sem).wait()
    n_chunks = E_PER_W // EC

    def chunk_body(ci, _):
        base = e0 + ci * EC
        gi = w * n_chunks + ci
        p0c = _sread(pcnt_v, gi)
        p1c = _sread(pcnt_v, gi + 1)

        def zero_row(r, __):
            for k in range(D // 16):
                acc_v[r, pl.ds(16 * k, 16)] = jnp.zeros((16,), jnp.float32)
            return 0
        lax.fori_loop(0, EC + 8, zero_row, 0, unroll=False)

        bb0 = (p0c // PBUF) * PBUF
        nb = jnp.where(p1c > p0c, (p1c - bb0 + PBUF - 1) // PBUF, 0)

        def blk(b, __):
            bb = pl.multiple_of(bb0 + b * PBUF, PBUF)
            c1 = pltpu.async_copy(pb_hbm.at[pl.ds(bb, PBUF)], pb_v.at[pl.ds(0, PBUF)], sem)
            c2 = pltpu.async_copy(pe_hbm.at[pl.ds(bb, PBUF)], pe_v.at[pl.ds(0, PBUF)], sem)
            c3 = pltpu.async_copy(pidx_hbm.at[pl.ds(bb, PBUF)], pidx_v.at[pl.ds(0, PBUF)], sem)
            c1.wait()
            c2.wait()
            c3.wait()
            pltpu.async_copy(zinit_hbm.at[pidx_v.at[pl.ds(0, PBUF)]], zrows_v, sem).wait()

            def pos(j, __):
                p = bb + j
                valid = (p >= p0c) & (p < p1c)
                le = jnp.where(valid, _sread(pb_v, j) - base, EC)
                pe = _sread(pe_v, j)
                for k in range(D // 16):
                    sl = pl.ds(16 * k, 16)
                    plsc.addupdate(acc_v.at[le, sl], zrows_v[j, sl] * pe)
                return 0
            lax.fori_loop(0, PBUF, pos, 0, unroll=False)
            return 0
        lax.fori_loop(0, nb, blk, 0, unroll=False)

        pltpu.async_copy(acc_v.at[pl.ds(0, EC)], zraw_hbm.at[pl.ds(base, EC)], sem).wait()
        return 0

    lax.fori_loop(0, n_chunks, chunk_body, 0, unroll=False)


_k1 = functools.partial(
    pl.kernel,
    out_type=jax.ShapeDtypeStruct((N_EDGES, D), jnp.float32),
    mesh=plsc.VectorSubcoreMesh(core_axis_name="c", subcore_axis_name="s", num_cores=2, num_subcores=16),
    scratch_types=[
        pltpu.VMEM((816,), jnp.int32),
        pltpu.VMEM((PBUF + 16,), jnp.int32),
        pltpu.VMEM((PBUF + 16,), jnp.float32),
        pltpu.VMEM((PBUF + 16,), jnp.int32),
        pltpu.VMEM((PBUF, D), jnp.float32),
        pltpu.VMEM((EC + 8, D), jnp.float32),
        pltpu.SemaphoreType.DMA,
    ],
)(_k1_body)


def _k3_body(hflat_hbm, eflat_hbm, src_hbm, dst_hbm, larr_hbm, agg_hbm,
             larr_v, sidx_v, didx_v, gidx_v, xbuf_v, ebuf_v, acc_sp, sem):
    c = lax.axis_index("c")
    s = lax.axis_index("s")
    pltpu.async_copy(larr_hbm, larr_v, sem).wait()
    lcbase = (larr_v[pl.ds(0, 16)][0] * 2 + c) * N_EDGES

    def zero_row(r, _):
        for k in range(HALF // 16):
            xbuf_v[r, pl.ds(16 * k, 16)] = jnp.zeros((16,), jnp.float32)
        return 0
    lax.fori_loop(0, K3_C, zero_row, 0, unroll=False)

    nloc = (NODE_NC - s + 15) // 16

    def zero_chunk(j, _):
        i = s + 16 * j
        pltpu.async_copy(xbuf_v, acc_sp.at[pl.ds(i * K3_C, K3_C)], sem).wait()
        return 0
    lax.fori_loop(0, nloc, zero_chunk, 0, unroll=False)
    plsc.subcore_barrier()

    ew0 = s * K3_EW

    def chunk(ci, _):
        eb = pl.multiple_of(ew0 + ci * K3_C, K3_C)
        c1 = pltpu.async_copy(src_hbm.at[pl.ds(eb, K3_C)], sidx_v, sem)
        c2 = pltpu.async_copy(dst_hbm.at[pl.ds(eb, K3_C)], didx_v, sem)
        c1.wait(); c2.wait()
        for k in range(K3_C // 16):
            sl = pl.ds(16 * k, 16)
            gidx_v[sl] = sidx_v[sl] + c * N_NODES
        pltpu.async_copy(hflat_hbm.at[gidx_v], xbuf_v, sem).wait()
        pltpu.async_copy(eflat_hbm.at[pl.ds(pl.multiple_of(lcbase + eb, K3_C), K3_C)], ebuf_v, sem).wait()

        def mrow(r, _):
            for k in range(HALF // 16):
                sl = pl.ds(16 * k, 16)
                xbuf_v[r, sl] = jnp.maximum(xbuf_v[r, sl] + ebuf_v[r, sl], 0.0)
            return 0
        lax.fori_loop(0, K3_C, mrow, 0, unroll=False)

        pltpu.async_copy(xbuf_v, acc_sp.at[didx_v], sem, add=True).wait()
        return 0

    lax.fori_loop(0, K3_NC, chunk, 0, unroll=False)
    plsc.subcore_barrier()

    def wb_chunk(j, _):
        i = s + 16 * j
        pltpu.async_copy(acc_sp.at[pl.ds(i * K3_C, K3_C)], xbuf_v, sem).wait()
        pltpu.async_copy(xbuf_v, agg_hbm.at[pl.ds(pl.multiple_of(c * N_NODES + i * K3_C, K3_C), K3_C)], sem).wait()
        return 0
    lax.fori_loop(0, nloc, wb_chunk, 0, unroll=False)


_k3 = functools.partial(
    pl.kernel,
    out_type=jax.ShapeDtypeStruct((2 * N_NODES, HALF), jnp.float32),
    mesh=plsc.VectorSubcoreMesh(core_axis_name="c", subcore_axis_name="s", num_cores=2, num_subcores=16),
    scratch_types=[
        pltpu.VMEM((16,), jnp.int32),
        pltpu.VMEM((K3_C,), jnp.int32),
        pltpu.VMEM((K3_C,), jnp.int32),
        pltpu.VMEM((K3_C,), jnp.int32),
        pltpu.VMEM((K3_C, HALF), jnp.float32),
        pltpu.VMEM((K3_C, HALF), jnp.float32),
        pltpu.VMEM_SHARED((N_NODES, HALF), jnp.float32),
        pltpu.SemaphoreType.DMA,
    ],
)(_k3_body)


K2_B = 640
K2_NB = N_EDGES // K2_B


def _k2_body(zraw_ref, zw_ref, zb_ref, zg1_ref, zbt1_ref, zg2_ref, zbt2_ref,
             we_ref, be_ref, e_ref):
    z = zraw_ref[...]
    z1 = jnp.maximum(z * (BN_C * zg1_ref[...]) + zbt1_ref[...], 0.0)
    t = jnp.dot(z1, zw_ref[...], preferred_element_type=jnp.float32) + zb_ref[...]
    z2 = jnp.maximum(t * (BN_C * zg2_ref[...]) + zbt2_ref[...], 0.0)
    for l in range(NUM_LAYERS):
        el = jnp.dot(z2, we_ref[l], preferred_element_type=jnp.float32) + be_ref[l]
        e_ref[2 * l] = el[:, :HALF]
        e_ref[2 * l + 1] = el[:, HALF:]


def _k2_call(zraw, zw, zb, zg1, zbt1, zg2, zbt2, we, be):
    full = lambda shape: pl.BlockSpec(shape, lambda i: (0,) * len(shape))
    return pl.pallas_call(
        _k2_body,
        grid=(K2_NB,),
        in_specs=[
            pl.BlockSpec((K2_B, D), lambda i: (i, 0)),
            full((D, D)), full((D,)), full((D,)), full((D,)),
            full((D,)), full((D,)),
            full((NUM_LAYERS, D, D)), full((NUM_LAYERS, D)),
        ],
        out_specs=pl.BlockSpec((2 * NUM_LAYERS, K2_B, HALF), lambda i: (0, i, 0)),
        out_shape=jax.ShapeDtypeStruct((2 * NUM_LAYERS, N_EDGES, HALF), jnp.float32),
    )(zraw, zw, zb, zg1, zbt1, zg2, zbt2, we, be)


K4_B = 400
K4_NB = N_NODES // K4_B


def _k4_body(h_ref, aggl_ref, aggr_ref, w1_ref, b1_ref, g1_ref, bt1_ref,
             w2_ref, b2_ref, g2_ref, bt2_ref, eps_ref, hfull_ref, hflat_ref):
    agg = jnp.concatenate([aggl_ref[...], aggr_ref[...]], axis=1)
    hin = eps_ref[0, 0] * h_ref[...] + agg
    t1 = jnp.dot(hin, w1_ref[...], preferred_element_type=jnp.float32) + b1_ref[...]
    h1 = jnp.maximum(t1 * (BN_C * g1_ref[...]) + bt1_ref[...], 0.0)
    t2 = jnp.dot(h1, w2_ref[...], preferred_element_type=jnp.float32) + b2_ref[...]
    h = jnp.maximum(t2 * (BN_C * g2_ref[...]) + bt2_ref[...], 0.0)
    hfull_ref[...] = h
    hflat_ref[0] = h[:, :HALF]
    hflat_ref[1] = h[:, HALF:]


def _k4_call(h, agg, w1, b1, g1, bt1, w2, b2, g2, bt2, eps_arr):
    full = lambda shape: pl.BlockSpec(shape, lambda i: (0,) * len(shape))
    return pl.pallas_call(
        _k4_body,
        grid=(K4_NB,),
        in_specs=[
            pl.BlockSpec((K4_B, D), lambda i: (i, 0)),
            pl.BlockSpec((K4_B, HALF), lambda i: (i, 0)),
            pl.BlockSpec((K4_B, HALF), lambda i: (i + K4_NB, 0)),
            full((D, D)), full((D,)), full((D,)), full((D,)),
            full((D, D)), full((D,)), full((D,)), full((D,)),
            full((8, HALF)),
        ],
        out_specs=[
            pl.BlockSpec((K4_B, D), lambda i: (i, 0)),
            pl.BlockSpec((2, K4_B, HALF), lambda i: (0, i, 0)),
        ],
        out_shape=[
            jax.ShapeDtypeStruct((N_NODES, D), jnp.float32),
            jax.ShapeDtypeStruct((2, N_NODES, HALF), jnp.float32),
        ],
    )(h, agg, agg, w1, b1, g1, bt1, w2, b2, g2, bt2, eps_arr)


K5_B = 400
K5_NB = N_NODES // K5_B


def _k5_body(batch_ref, h1_ref, h2_ref, h3_ref, h4_ref, w1_ref, b1_ref,
             bg_ref, bb_ref, w2_ref, b2_ref, out_ref, pooled_sc, cnt_sc):
    i = pl.program_id(0)

    @pl.when(i == 0)
    def _():
        pooled_sc[...] = jnp.zeros_like(pooled_sc)
        cnt_sc[...] = jnp.zeros_like(cnt_sc)

    b = batch_ref[pl.ds(i, 1), 0, :]
    gids = lax.broadcasted_iota(jnp.int32, (N_GRAPHS, K5_B), 0)
    oh = (gids == b).astype(jnp.float32)
    hcat = jnp.concatenate(
        [h1_ref[...], h2_ref[...], h3_ref[...], h4_ref[...]], axis=1)
    pooled_sc[...] += jnp.dot(oh, hcat, preferred_element_type=jnp.float32)
    rs = jnp.sum(oh, axis=1, keepdims=True)
    cnt_sc[...] += jnp.broadcast_to(rs, (N_GRAPHS, HALF))

    @pl.when(i == K5_NB - 1)
    def _():
        cnt = jnp.maximum(cnt_sc[:, :1], 1.0)
        gm = pooled_sc[...] / cnt
        g = jnp.dot(gm, w1_ref[...], preferred_element_type=jnp.float32) + b1_ref[...]
        g = g * (BN_C * bg_ref[...]) + bb_ref[...]
        g = jnp.maximum(g, 0.0)
        logits = jnp.dot(g, w2_ref[...], preferred_element_type=jnp.float32) + b2_ref[...]
        col = lax.broadcasted_iota(jnp.int32, (N_GRAPHS, HALF), 1)
        masked = jnp.where(col < 16, logits, -3e38)
        m = jnp.max(masked, axis=1, keepdims=True)
        ex = jnp.where(col < 16, jnp.exp(logits - m), 0.0)
        lse = jnp.log(jnp.sum(ex, axis=1, keepdims=True))
        out_ref[...] = logits - m - lse


def _k5_call(batch3, h1, h2, h3, h4, w1, b1, bg, bb, w2p, b2p):
    full = lambda shape: pl.BlockSpec(shape, lambda i: (0,) * len(shape))
    hspec = pl.BlockSpec((K5_B, D), lambda i: (i, 0))
    return pl.pallas_call(
        _k5_body,
        grid=(K5_NB,),
        in_specs=[
            full((K5_NB, 1, K5_B)),
            hspec, hspec, hspec, hspec,
            full((NUM_LAYERS * D, D)), full((D,)),
            full((D,)), full((D,)),
            full((D, HALF)), full((HALF,)),
        ],
        out_specs=pl.BlockSpec((N_GRAPHS, HALF), lambda i: (0, 0)),
        out_shape=jax.ShapeDtypeStruct((N_GRAPHS, HALF), jnp.float32),
        scratch_shapes=[
            pltpu.VMEM((N_GRAPHS, NUM_LAYERS * D), jnp.float32),
            pltpu.VMEM((N_GRAPHS, HALF), jnp.float32),
        ],
    )(batch3, h1, h2, h3, h4, w1, b1, bg, bb, w2p, b2p)


def kernel(x, pos_enc, params, edge_index, batch, pos_index, pos_batch):
    src = edge_index[0]
    dst = edge_index[1]
    bounds = jnp.arange(0, N_EDGES + 1, EC, dtype=jnp.int32)
    pcnt = jnp.searchsorted(pos_batch, bounds).astype(jnp.int32)
    pcnt = jnp.pad(pcnt, (0, 816 - pcnt.shape[0]))

    zraw = _k1(params['z_init'], pos_index, pos_enc, pos_batch, pcnt)

    convs = [params['conv1']] + list(params['convs'])
    we = jnp.stack([cp['we'] for cp in convs])
    be = jnp.stack([cp['be'] for cp in convs])
    e = _k2_call(zraw, params['z_w'], params['z_b'], params['z_g1'],
                 params['z_bt1'], params['z_g2'], params['z_bt2'], we, be)
    eflat = e.reshape(2 * NUM_LAYERS * N_EDGES, HALF)

    h = x
    hflat = jnp.concatenate([x[:, :HALF], x[:, HALF:]], axis=0)
    hs = []
    for l, cp in enumerate(convs):
        larr = jnp.full((16,), l, jnp.int32)
        agg = _k3(hflat, eflat, src, dst, larr)
        eps_arr = jnp.full((8, HALF), 1.0 + cp['eps'], jnp.float32)
        h, hfl2 = _k4_call(h, agg, cp['w1'], cp['b1'], cp['g1'], cp['bt1'],
                           cp['w2'], cp['b2'], cp['g2'], cp['bt2'], eps_arr)
        hflat = hfl2.reshape(2 * N_NODES, HALF)
        hs.append(h)

    batch3 = batch.reshape(K5_NB, 1, K5_B)
    w2p = jnp.pad(params['lin2_w'], ((0, 0), (0, HALF - 16)))
    b2p = jnp.pad(params['lin2_b'], (0, HALF - 16))
    out = _k5_call(batch3, hs[0], hs[1], hs[2], hs[3],
                   params['lin1_w'], params['lin1_b'],
                   params['bn_g'], params['bn_b'], w2p, b2p)
    return out[:, :16]


# speedup vs baseline: 1.7058x; 1.0244x over previous
import functools

import jax
import jax.numpy as jnp
from jax import lax
from jax.experimental import pallas as pl
from jax.experimental.pallas import tpu as pltpu
from jax.experimental.pallas import tpu_sc as plsc

N_NODES = 10000
N_EDGES = 160000
D = 256
HALF = 128
NUM_LAYERS = 4
Z_IN = 1800
P_POS = 320000
N_GRAPHS = 128
BN_C = (1.0 + 1e-5) ** -0.5

def _sread(ref, i):
    return ref[pl.ds(i, 16)][0]


NW = 32
E_PER_W = N_EDGES // NW
EC = 200
PBUF = 128
K3_EW = N_EDGES // 16
K3_C = 80
K3_NC = K3_EW // K3_C
NODE_NC = N_NODES // K3_C


def _k1_body(zinit_hbm, pidx_hbm, pe_hbm, pb_hbm, pcnt_hbm, zraw_hbm,
             pcnt_v, pb_a, pe_a, pidx_a, pb_b, pe_b, pidx_b,
             zrows_a, zrows_b, acc_v,
             sem_ia, sem_ib, sem_ga, sem_gb, sem_f):
    c = lax.axis_index("c")
    s = lax.axis_index("s")
    w = s * 2 + c
    e0 = w * E_PER_W
    pltpu.async_copy(pcnt_hbm, pcnt_v, sem_ia).wait()
    n_chunks = E_PER_W // EC
    pmax = P_POS - PBUF

    def issue_idx(bb, pbv, pev, piv, sem):
        c1 = pltpu.async_copy(pb_hbm.at[pl.ds(bb, PBUF)], pbv.at[pl.ds(0, PBUF)], sem)
        c2 = pltpu.async_copy(pe_hbm.at[pl.ds(bb, PBUF)], pev.at[pl.ds(0, PBUF)], sem)
        c3 = pltpu.async_copy(pidx_hbm.at[pl.ds(bb, PBUF)], piv.at[pl.ds(0, PBUF)], sem)
        c1.wait()
        c2.wait()
        c3.wait()

    def issue_gather(piv, zrv, sem):
        pltpu.async_copy(zinit_hbm.at[piv.at[pl.ds(0, PBUF)]], zrv, sem)

    def wait_gather(piv, zrv, sem):
        pltpu.make_async_copy(zinit_hbm.at[piv.at[pl.ds(0, PBUF)]], zrv, sem).wait()

    def chunk_body(ci, _):
        base = e0 + ci * EC
        gi = w * n_chunks + ci
        p0c = _sread(pcnt_v, gi)
        p1c = _sread(pcnt_v, gi + 1)
        bb0 = (p0c // PBUF) * PBUF
        nb = jnp.where(p1c > p0c, (p1c - bb0 + PBUF - 1) // PBUF, 0)

        def bidx(k):
            return pl.multiple_of(jnp.minimum(bb0 + k * PBUF, pmax), PBUF)

        issue_idx(bidx(0), pb_a, pe_a, pidx_a, sem_ia)
        issue_gather(pidx_a, zrows_a, sem_ga)

        def zero_row(r, __):
            for k in range(D // 16):
                acc_v[r, pl.ds(16 * k, 16)] = jnp.zeros((16,), jnp.float32)
            return 0
        lax.fori_loop(0, EC + 8, zero_row, 0, unroll=False)

        def pos_loop(bb, be, pbv, pev, zrv):
            def pos(j, __):
                p = bb + j
                valid = (p >= p0c) & (p < p1c) & (be < nb)
                le = jnp.where(valid, _sread(pbv, j) - base, EC)
                pe = _sread(pev, j)
                for k in range(D // 16):
                    sl = pl.ds(16 * k, 16)
                    plsc.addupdate(acc_v.at[le, sl], zrv[j, sl] * pe)
                return 0
            lax.fori_loop(0, PBUF, pos, 0, unroll=False)

        def pair(bp, __):
            e = 2 * bp
            issue_idx(bidx(e + 1), pb_b, pe_b, pidx_b, sem_ib)
            issue_gather(pidx_b, zrows_b, sem_gb)
            wait_gather(pidx_a, zrows_a, sem_ga)
            pos_loop(bidx(e), e, pb_a, pe_a, zrows_a)
            issue_idx(bidx(e + 2), pb_a, pe_a, pidx_a, sem_ia)
            issue_gather(pidx_a, zrows_a, sem_ga)
            wait_gather(pidx_b, zrows_b, sem_gb)
            pos_loop(bidx(e + 1), e + 1, pb_b, pe_b, zrows_b)
            return 0
        lax.fori_loop(0, (nb + 1) // 2, pair, 0, unroll=False)

        wait_gather(pidx_a, zrows_a, sem_ga)
        pltpu.async_copy(acc_v.at[pl.ds(0, EC)], zraw_hbm.at[pl.ds(base, EC)], sem_f).wait()
        return 0

    lax.fori_loop(0, n_chunks, chunk_body, 0, unroll=False)


_k1 = functools.partial(
    pl.kernel,
    out_type=jax.ShapeDtypeStruct((N_EDGES, D), jnp.float32),
    mesh=plsc.VectorSubcoreMesh(core_axis_name="c", subcore_axis_name="s", num_cores=2, num_subcores=16),
    scratch_types=[
        pltpu.VMEM((816,), jnp.int32),
        pltpu.VMEM((PBUF + 16,), jnp.int32),
        pltpu.VMEM((PBUF + 16,), jnp.float32),
        pltpu.VMEM((PBUF + 16,), jnp.int32),
        pltpu.VMEM((PBUF + 16,), jnp.int32),
        pltpu.VMEM((PBUF + 16,), jnp.float32),
        pltpu.VMEM((PBUF + 16,), jnp.int32),
        pltpu.VMEM((PBUF, D), jnp.float32),
        pltpu.VMEM((PBUF, D), jnp.float32),
        pltpu.VMEM((EC + 8, D), jnp.float32),
        pltpu.SemaphoreType.DMA,
        pltpu.SemaphoreType.DMA,
        pltpu.SemaphoreType.DMA,
        pltpu.SemaphoreType.DMA,
        pltpu.SemaphoreType.DMA,
    ],
)(_k1_body)


def _k3_body(hflat_hbm, eflat_hbm, src_hbm, dst_hbm, larr_hbm, agg_hbm,
             larr_v, sidx_v, didx_v, gidx_v, xbuf_v, ebuf_v, acc_sp, sem):
    c = lax.axis_index("c")
    s = lax.axis_index("s")
    pltpu.async_copy(larr_hbm, larr_v, sem).wait()
    lcbase = (larr_v[pl.ds(0, 16)][0] * 2 + c) * N_EDGES

    def zero_row(r, _):
        for k in range(HALF // 16):
            xbuf_v[r, pl.ds(16 * k, 16)] = jnp.zeros((16,), jnp.float32)
        return 0
    lax.fori_loop(0, K3_C, zero_row, 0, unroll=False)

    nloc = (NODE_NC - s + 15) // 16

    def zero_chunk(j, _):
        i = s + 16 * j
        pltpu.async_copy(xbuf_v, acc_sp.at[pl.ds(i * K3_C, K3_C)], sem).wait()
        return 0
    lax.fori_loop(0, nloc, zero_chunk, 0, unroll=False)
    plsc.subcore_barrier()

    ew0 = s * K3_EW

    def chunk(ci, _):
        eb = pl.multiple_of(ew0 + ci * K3_C, K3_C)
        c1 = pltpu.async_copy(src_hbm.at[pl.ds(eb, K3_C)], sidx_v, sem)
        c2 = pltpu.async_copy(dst_hbm.at[pl.ds(eb, K3_C)], didx_v, sem)
        c1.wait()
        c2.wait()
        for k in range(K3_C // 16):
            sl = pl.ds(16 * k, 16)
            gidx_v[sl] = sidx_v[sl] + c * N_NODES
        pltpu.async_copy(hflat_hbm.at[gidx_v], xbuf_v, sem).wait()
        pltpu.async_copy(eflat_hbm.at[pl.ds(pl.multiple_of(lcbase + eb, K3_C), K3_C)], ebuf_v, sem).wait()

        def mrow(r, _):
            for k in range(HALF // 16):
                sl = pl.ds(16 * k, 16)
                xbuf_v[r, sl] = jnp.maximum(xbuf_v[r, sl] + ebuf_v[r, sl], 0.0)
            return 0
        lax.fori_loop(0, K3_C, mrow, 0, unroll=False)

        pltpu.async_copy(xbuf_v, acc_sp.at[didx_v], sem, add=True).wait()
        return 0

    lax.fori_loop(0, K3_NC, chunk, 0, unroll=False)
    plsc.subcore_barrier()

    def wb_chunk(j, _):
        i = s + 16 * j
        pltpu.async_copy(acc_sp.at[pl.ds(i * K3_C, K3_C)], xbuf_v, sem).wait()
        pltpu.async_copy(xbuf_v, agg_hbm.at[pl.ds(pl.multiple_of(c * N_NODES + i * K3_C, K3_C), K3_C)], sem).wait()
        return 0
    lax.fori_loop(0, nloc, wb_chunk, 0, unroll=False)


_k3 = functools.partial(
    pl.kernel,
    out_type=jax.ShapeDtypeStruct((2 * N_NODES, HALF), jnp.float32),
    mesh=plsc.VectorSubcoreMesh(core_axis_name="c", subcore_axis_name="s", num_cores=2, num_subcores=16),
    scratch_types=[
        pltpu.VMEM((16,), jnp.int32),
        pltpu.VMEM((K3_C,), jnp.int32),
        pltpu.VMEM((K3_C,), jnp.int32),
        pltpu.VMEM((K3_C,), jnp.int32),
        pltpu.VMEM((K3_C, HALF), jnp.float32),
        pltpu.VMEM((K3_C, HALF), jnp.float32),
        pltpu.VMEM_SHARED((N_NODES, HALF), jnp.float32),
        pltpu.SemaphoreType.DMA,
    ],
)(_k3_body)


K2_B = 640
K2_NB = N_EDGES // K2_B


def _k2_body(zraw_ref, zw_ref, zb_ref, zg1_ref, zbt1_ref, zg2_ref, zbt2_ref,
             we_ref, be_ref, e_ref):
    z = zraw_ref[...]
    z1 = jnp.maximum(z * (BN_C * zg1_ref[...]) + zbt1_ref[...], 0.0)
    t = jnp.dot(z1, zw_ref[...], preferred_element_type=jnp.float32) + zb_ref[...]
    z2 = jnp.maximum(t * (BN_C * zg2_ref[...]) + zbt2_ref[...], 0.0)
    for l in range(NUM_LAYERS):
        el = jnp.dot(z2, we_ref[l], preferred_element_type=jnp.float32) + be_ref[l]
        e_ref[2 * l] = el[:, :HALF]
        e_ref[2 * l + 1] = el[:, HALF:]


def _k2_call(zraw, zw, zb, zg1, zbt1, zg2, zbt2, we, be):
    full = lambda shape: pl.BlockSpec(shape, lambda i: (0,) * len(shape))
    return pl.pallas_call(
        _k2_body,
        grid=(K2_NB,),
        in_specs=[
            pl.BlockSpec((K2_B, D), lambda i: (i, 0)),
            full((D, D)), full((D,)), full((D,)), full((D,)),
            full((D,)), full((D,)),
            full((NUM_LAYERS, D, D)), full((NUM_LAYERS, D)),
        ],
        out_specs=pl.BlockSpec((2 * NUM_LAYERS, K2_B, HALF), lambda i: (0, i, 0)),
        out_shape=jax.ShapeDtypeStruct((2 * NUM_LAYERS, N_EDGES, HALF), jnp.float32),
    )(zraw, zw, zb, zg1, zbt1, zg2, zbt2, we, be)


K4_B = 400
K4_NB = N_NODES // K4_B


def _k4_body(h_ref, aggl_ref, aggr_ref, w1_ref, b1_ref, g1_ref, bt1_ref,
             w2_ref, b2_ref, g2_ref, bt2_ref, eps_ref, hfull_ref, hflat_ref):
    agg = jnp.concatenate([aggl_ref[...], aggr_ref[...]], axis=1)
    hin = eps_ref[0, 0] * h_ref[...] + agg
    t1 = jnp.dot(hin, w1_ref[...], preferred_element_type=jnp.float32) + b1_ref[...]
    h1 = jnp.maximum(t1 * (BN_C * g1_ref[...]) + bt1_ref[...], 0.0)
    t2 = jnp.dot(h1, w2_ref[...], preferred_element_type=jnp.float32) + b2_ref[...]
    h = jnp.maximum(t2 * (BN_C * g2_ref[...]) + bt2_ref[...], 0.0)
    hfull_ref[...] = h
    hflat_ref[0] = h[:, :HALF]
    hflat_ref[1] = h[:, HALF:]


def _k4_call(h, agg, w1, b1, g1, bt1, w2, b2, g2, bt2, eps_arr):
    full = lambda shape: pl.BlockSpec(shape, lambda i: (0,) * len(shape))
    return pl.pallas_call(
        _k4_body,
        grid=(K4_NB,),
        in_specs=[
            pl.BlockSpec((K4_B, D), lambda i: (i, 0)),
            pl.BlockSpec((K4_B, HALF), lambda i: (i, 0)),
            pl.BlockSpec((K4_B, HALF), lambda i: (i + K4_NB, 0)),
            full((D, D)), full((D,)), full((D,)), full((D,)),
            full((D, D)), full((D,)), full((D,)), full((D,)),
            full((8, HALF)),
        ],
        out_specs=[
            pl.BlockSpec((K4_B, D), lambda i: (i, 0)),
            pl.BlockSpec((2, K4_B, HALF), lambda i: (0, i, 0)),
        ],
        out_shape=[
            jax.ShapeDtypeStruct((N_NODES, D), jnp.float32),
            jax.ShapeDtypeStruct((2, N_NODES, HALF), jnp.float32),
        ],
    )(h, agg, agg, w1, b1, g1, bt1, w2, b2, g2, bt2, eps_arr)


K5_B = 400
K5_NB = N_NODES // K5_B


def _k5_body(batch_ref, h1_ref, h2_ref, h3_ref, h4_ref, w1_ref, b1_ref,
             bg_ref, bb_ref, w2_ref, b2_ref, out_ref, pooled_sc, cnt_sc):
    i = pl.program_id(0)

    @pl.when(i == 0)
    def _():
        pooled_sc[...] = jnp.zeros_like(pooled_sc)
        cnt_sc[...] = jnp.zeros_like(cnt_sc)

    b = batch_ref[pl.ds(i, 1), 0, :]
    gids = lax.broadcasted_iota(jnp.int32, (N_GRAPHS, K5_B), 0)
    oh = (gids == b).astype(jnp.float32)
    hcat = jnp.concatenate(
        [h1_ref[...], h2_ref[...], h3_ref[...], h4_ref[...]], axis=1)
    pooled_sc[...] += jnp.dot(oh, hcat, preferred_element_type=jnp.float32)
    rs = jnp.sum(oh, axis=1, keepdims=True)
    cnt_sc[...] += jnp.broadcast_to(rs, (N_GRAPHS, HALF))

    @pl.when(i == K5_NB - 1)
    def _():
        cnt = jnp.maximum(cnt_sc[:, :1], 1.0)
        gm = pooled_sc[...] / cnt
        g = jnp.dot(gm, w1_ref[...], preferred_element_type=jnp.float32) + b1_ref[...]
        g = g * (BN_C * bg_ref[...]) + bb_ref[...]
        g = jnp.maximum(g, 0.0)
        logits = jnp.dot(g, w2_ref[...], preferred_element_type=jnp.float32) + b2_ref[...]
        col = lax.broadcasted_iota(jnp.int32, (N_GRAPHS, HALF), 1)
        masked = jnp.where(col < 16, logits, -3e38)
        m = jnp.max(masked, axis=1, keepdims=True)
        ex = jnp.where(col < 16, jnp.exp(logits - m), 0.0)
        lse = jnp.log(jnp.sum(ex, axis=1, keepdims=True))
        out_ref[...] = logits - m - lse


def _k5_call(batch3, h1, h2, h3, h4, w1, b1, bg, bb, w2p, b2p):
    full = lambda shape: pl.BlockSpec(shape, lambda i: (0,) * len(shape))
    hspec = pl.BlockSpec((K5_B, D), lambda i: (i, 0))
    return pl.pallas_call(
        _k5_body,
        grid=(K5_NB,),
        in_specs=[
            full((K5_NB, 1, K5_B)),
            hspec, hspec, hspec, hspec,
            full((NUM_LAYERS * D, D)), full((D,)),
            full((D,)), full((D,)),
            full((D, HALF)), full((HALF,)),
        ],
        out_specs=pl.BlockSpec((N_GRAPHS, HALF), lambda i: (0, 0)),
        out_shape=jax.ShapeDtypeStruct((N_GRAPHS, HALF), jnp.float32),
        scratch_shapes=[
            pltpu.VMEM((N_GRAPHS, NUM_LAYERS * D), jnp.float32),
            pltpu.VMEM((N_GRAPHS, HALF), jnp.float32),
        ],
    )(batch3, h1, h2, h3, h4, w1, b1, bg, bb, w2p, b2p)


def kernel(x, pos_enc, params, edge_index, batch, pos_index, pos_batch):
    src = edge_index[0]
    dst = edge_index[1]
    bounds = jnp.arange(0, N_EDGES + 1, EC, dtype=jnp.int32)
    pcnt = jnp.searchsorted(pos_batch, bounds).astype(jnp.int32)
    pcnt = jnp.pad(pcnt, (0, 816 - pcnt.shape[0]))

    zraw = _k1(params['z_init'], pos_index, pos_enc, pos_batch, pcnt)

    convs = [params['conv1']] + list(params['convs'])
    we = jnp.stack([cp['we'] for cp in convs])
    be = jnp.stack([cp['be'] for cp in convs])
    e = _k2_call(zraw, params['z_w'], params['z_b'], params['z_g1'],
                 params['z_bt1'], params['z_g2'], params['z_bt2'], we, be)
    eflat = e.reshape(2 * NUM_LAYERS * N_EDGES, HALF)

    h = x
    hflat = jnp.concatenate([x[:, :HALF], x[:, HALF:]], axis=0)
    hs = []
    for l, cp in enumerate(convs):
        larr = jnp.full((16,), l, jnp.int32)
        agg = _k3(hflat, eflat, src, dst, larr)
        eps_arr = jnp.full((8, HALF), 1.0 + cp['eps'], jnp.float32)
        h, hfl2 = _k4_call(h, agg, cp['w1'], cp['b1'], cp['g1'], cp['bt1'],
                           cp['w2'], cp['b2'], cp['g2'], cp['bt2'], eps_arr)
        hflat = hfl2.reshape(2 * N_NODES, HALF)
        hs.append(h)

    batch3 = batch.reshape(K5_NB, 1, K5_B)
    w2p = jnp.pad(params['lin2_w'], ((0, 0), (0, HALF - 16)))
    b2p = jnp.pad(params['lin2_b'], (0, HALF - 16))
    out = _k5_call(batch3, hs[0], hs[1], hs[2], hs[3],
                   params['lin1_w'], params['lin1_b'],
                   params['bn_g'], params['bn_b'], w2p, b2p)
    return out[:, :16]


# speedup vs baseline: 2.1346x; 1.2514x over previous
import functools

import jax
import jax.numpy as jnp
from jax import lax
from jax.experimental import pallas as pl
from jax.experimental.pallas import tpu as pltpu
from jax.experimental.pallas import tpu_sc as plsc

N_NODES = 10000
N_EDGES = 160000
D = 256
HALF = 128
NUM_LAYERS = 4
Z_IN = 1800
P_POS = 320000
N_GRAPHS = 128
BN_C = (1.0 + 1e-5) ** -0.5

def _sread(ref, i):
    return ref[pl.ds(i, 16)][0]


NW = 32
E_PER_W = N_EDGES // NW
EC = 200
PBUF = 128
K3_EW = N_EDGES // 16
K3_C = 80
K3_NC = K3_EW // K3_C
NODE_NC = N_NODES // K3_C


def _k1_body(zinit_hbm, pidx_hbm, pe_hbm, pb_hbm, pcnt_hbm, zraw_hbm,
             pcnt_v, pb_a, pe_a, pidx_a, pb_b, pe_b, pidx_b,
             zrows_a, zrows_b, acc_v,
             sem_ia, sem_ib, sem_ga, sem_gb, sem_f):
    c = lax.axis_index("c")
    s = lax.axis_index("s")
    w = s * 2 + c
    e0 = w * E_PER_W
    pltpu.async_copy(pcnt_hbm, pcnt_v, sem_ia).wait()
    n_chunks = E_PER_W // EC
    pmax = P_POS - PBUF

    def issue_idx(bb, pbv, pev, piv, sem):
        c1 = pltpu.async_copy(pb_hbm.at[pl.ds(bb, PBUF)], pbv.at[pl.ds(0, PBUF)], sem)
        c2 = pltpu.async_copy(pe_hbm.at[pl.ds(bb, PBUF)], pev.at[pl.ds(0, PBUF)], sem)
        c3 = pltpu.async_copy(pidx_hbm.at[pl.ds(bb, PBUF)], piv.at[pl.ds(0, PBUF)], sem)
        c1.wait()
        c2.wait()
        c3.wait()

    def issue_gather(piv, zrv, sem):
        pltpu.async_copy(zinit_hbm.at[piv.at[pl.ds(0, PBUF)]], zrv, sem)

    def wait_gather(piv, zrv, sem):
        pltpu.make_async_copy(zinit_hbm.at[piv.at[pl.ds(0, PBUF)]], zrv, sem).wait()

    def chunk_body(ci, _):
        base = e0 + ci * EC
        gi = w * n_chunks + ci
        p0c = _sread(pcnt_v, gi)
        p1c = _sread(pcnt_v, gi + 1)
        bb0 = (p0c // PBUF) * PBUF
        nb = jnp.where(p1c > p0c, (p1c - bb0 + PBUF - 1) // PBUF, 0)

        def bidx(k):
            return pl.multiple_of(jnp.minimum(bb0 + k * PBUF, pmax), PBUF)

        issue_idx(bidx(0), pb_a, pe_a, pidx_a, sem_ia)
        issue_gather(pidx_a, zrows_a, sem_ga)

        def zero_row(r, __):
            for k in range(D // 16):
                acc_v[r, pl.ds(16 * k, 16)] = jnp.zeros((16,), jnp.float32)
            return 0
        lax.fori_loop(0, EC + 8, zero_row, 0, unroll=False)

        def pos_loop(bb, be, pbv, pev, zrv):
            def pos(j, __):
                p = bb + j
                valid = (p >= p0c) & (p < p1c) & (be < nb)
                le = jnp.where(valid, _sread(pbv, j) - base, EC)
                pe = _sread(pev, j)
                vals = [zrv[j, pl.ds(16 * k, 16)] * pe for k in range(D // 16)]
                for k in range(D // 16):
                    plsc.addupdate(acc_v.at[le, pl.ds(16 * k, 16)], vals[k])
                return 0
            lax.fori_loop(0, PBUF, pos, 0, unroll=False)

        def pair(bp, __):
            e = 2 * bp
            issue_idx(bidx(e + 1), pb_b, pe_b, pidx_b, sem_ib)
            issue_gather(pidx_b, zrows_b, sem_gb)
            wait_gather(pidx_a, zrows_a, sem_ga)
            pos_loop(bidx(e), e, pb_a, pe_a, zrows_a)
            issue_idx(bidx(e + 2), pb_a, pe_a, pidx_a, sem_ia)
            issue_gather(pidx_a, zrows_a, sem_ga)
            wait_gather(pidx_b, zrows_b, sem_gb)
            pos_loop(bidx(e + 1), e + 1, pb_b, pe_b, zrows_b)
            return 0
        lax.fori_loop(0, (nb + 1) // 2, pair, 0, unroll=False)

        wait_gather(pidx_a, zrows_a, sem_ga)
        pltpu.async_copy(acc_v.at[pl.ds(0, EC)], zraw_hbm.at[pl.ds(base, EC)], sem_f).wait()
        return 0

    lax.fori_loop(0, n_chunks, chunk_body, 0, unroll=False)


_k1 = functools.partial(
    pl.kernel,
    out_type=jax.ShapeDtypeStruct((N_EDGES, D), jnp.float32),
    mesh=plsc.VectorSubcoreMesh(core_axis_name="c", subcore_axis_name="s", num_cores=2, num_subcores=16),
    scratch_types=[
        pltpu.VMEM((816,), jnp.int32),
        pltpu.VMEM((PBUF + 16,), jnp.int32),
        pltpu.VMEM((PBUF + 16,), jnp.float32),
        pltpu.VMEM((PBUF + 16,), jnp.int32),
        pltpu.VMEM((PBUF + 16,), jnp.int32),
        pltpu.VMEM((PBUF + 16,), jnp.float32),
        pltpu.VMEM((PBUF + 16,), jnp.int32),
        pltpu.VMEM((PBUF, D), jnp.float32),
        pltpu.VMEM((PBUF, D), jnp.float32),
        pltpu.VMEM((EC + 8, D), jnp.float32),
        pltpu.SemaphoreType.DMA,
        pltpu.SemaphoreType.DMA,
        pltpu.SemaphoreType.DMA,
        pltpu.SemaphoreType.DMA,
        pltpu.SemaphoreType.DMA,
    ],
)(_k1_body)


def _k3_body(hflat_hbm, eflat_hbm, src_hbm, dst_hbm, larr_hbm, agg_hbm,
             larr_v, sidx_v, didx_v, gidx_v, xbuf_v, ebuf_v, acc_sp, sem):
    c = lax.axis_index("c")
    s = lax.axis_index("s")
    pltpu.async_copy(larr_hbm, larr_v, sem).wait()
    lcbase = (larr_v[pl.ds(0, 16)][0] * 2 + c) * N_EDGES

    def zero_row(r, _):
        for k in range(HALF // 16):
            xbuf_v[r, pl.ds(16 * k, 16)] = jnp.zeros((16,), jnp.float32)
        return 0
    lax.fori_loop(0, K3_C, zero_row, 0, unroll=False)

    nloc = (NODE_NC - s + 15) // 16

    def zero_chunk(j, _):
        i = s + 16 * j
        pltpu.async_copy(xbuf_v, acc_sp.at[pl.ds(i * K3_C, K3_C)], sem).wait()
        return 0
    lax.fori_loop(0, nloc, zero_chunk, 0, unroll=False)
    plsc.subcore_barrier()

    ew0 = s * K3_EW

    def chunk(ci, _):
        eb = pl.multiple_of(ew0 + ci * K3_C, K3_C)
        c1 = pltpu.async_copy(src_hbm.at[pl.ds(eb, K3_C)], sidx_v, sem)
        c2 = pltpu.async_copy(dst_hbm.at[pl.ds(eb, K3_C)], didx_v, sem)
        c1.wait()
        c2.wait()
        for k in range(K3_C // 16):
            sl = pl.ds(16 * k, 16)
            gidx_v[sl] = sidx_v[sl] + c * N_NODES
        pltpu.async_copy(hflat_hbm.at[gidx_v], xbuf_v, sem).wait()
        pltpu.async_copy(eflat_hbm.at[pl.ds(pl.multiple_of(lcbase + eb, K3_C), K3_C)], ebuf_v, sem).wait()

        def mrow(r, _):
            for k in range(HALF // 16):
                sl = pl.ds(16 * k, 16)
                xbuf_v[r, sl] = jnp.maximum(xbuf_v[r, sl] + ebuf_v[r, sl], 0.0)
            return 0
        lax.fori_loop(0, K3_C, mrow, 0, unroll=False)

        pltpu.async_copy(xbuf_v, acc_sp.at[didx_v], sem, add=True).wait()
        return 0

    lax.fori_loop(0, K3_NC, chunk, 0, unroll=False)
    plsc.subcore_barrier()

    def wb_chunk(j, _):
        i = s + 16 * j
        pltpu.async_copy(acc_sp.at[pl.ds(i * K3_C, K3_C)], xbuf_v, sem).wait()
        pltpu.async_copy(xbuf_v, agg_hbm.at[pl.ds(pl.multiple_of(c * N_NODES + i * K3_C, K3_C), K3_C)], sem).wait()
        return 0
    lax.fori_loop(0, nloc, wb_chunk, 0, unroll=False)


_k3 = functools.partial(
    pl.kernel,
    out_type=jax.ShapeDtypeStruct((2 * N_NODES, HALF), jnp.float32),
    mesh=plsc.VectorSubcoreMesh(core_axis_name="c", subcore_axis_name="s", num_cores=2, num_subcores=16),
    scratch_types=[
        pltpu.VMEM((16,), jnp.int32),
        pltpu.VMEM((K3_C,), jnp.int32),
        pltpu.VMEM((K3_C,), jnp.int32),
        pltpu.VMEM((K3_C,), jnp.int32),
        pltpu.VMEM((K3_C, HALF), jnp.float32),
        pltpu.VMEM((K3_C, HALF), jnp.float32),
        pltpu.VMEM_SHARED((N_NODES, HALF), jnp.float32),
        pltpu.SemaphoreType.DMA,
    ],
)(_k3_body)


K2_B = 640
K2_NB = N_EDGES // K2_B


def _k2_body(zraw_ref, zw_ref, zb_ref, zg1_ref, zbt1_ref, zg2_ref, zbt2_ref,
             we_ref, be_ref, e_ref):
    z = zraw_ref[...]
    z1 = jnp.maximum(z * (BN_C * zg1_ref[...]) + zbt1_ref[...], 0.0)
    t = jnp.dot(z1, zw_ref[...], preferred_element_type=jnp.float32) + zb_ref[...]
    z2 = jnp.maximum(t * (BN_C * zg2_ref[...]) + zbt2_ref[...], 0.0)
    for l in range(NUM_LAYERS):
        el = jnp.dot(z2, we_ref[l], preferred_element_type=jnp.float32) + be_ref[l]
        e_ref[2 * l] = el[:, :HALF]
        e_ref[2 * l + 1] = el[:, HALF:]


def _k2_call(zraw, zw, zb, zg1, zbt1, zg2, zbt2, we, be):
    full = lambda shape: pl.BlockSpec(shape, lambda i: (0,) * len(shape))
    return pl.pallas_call(
        _k2_body,
        grid=(K2_NB,),
        in_specs=[
            pl.BlockSpec((K2_B, D), lambda i: (i, 0)),
            full((D, D)), full((D,)), full((D,)), full((D,)),
            full((D,)), full((D,)),
            full((NUM_LAYERS, D, D)), full((NUM_LAYERS, D)),
        ],
        out_specs=pl.BlockSpec((2 * NUM_LAYERS, K2_B, HALF), lambda i: (0, i, 0)),
        out_shape=jax.ShapeDtypeStruct((2 * NUM_LAYERS, N_EDGES, HALF), jnp.float32),
    )(zraw, zw, zb, zg1, zbt1, zg2, zbt2, we, be)


K4_B = 400
K4_NB = N_NODES // K4_B


def _k4_body(h_ref, aggl_ref, aggr_ref, w1_ref, b1_ref, g1_ref, bt1_ref,
             w2_ref, b2_ref, g2_ref, bt2_ref, eps_ref, hfull_ref, hflat_ref):
    agg = jnp.concatenate([aggl_ref[...], aggr_ref[...]], axis=1)
    hin = eps_ref[0, 0] * h_ref[...] + agg
    t1 = jnp.dot(hin, w1_ref[...], preferred_element_type=jnp.float32) + b1_ref[...]
    h1 = jnp.maximum(t1 * (BN_C * g1_ref[...]) + bt1_ref[...], 0.0)
    t2 = jnp.dot(h1, w2_ref[...], preferred_element_type=jnp.float32) + b2_ref[...]
    h = jnp.maximum(t2 * (BN_C * g2_ref[...]) + bt2_ref[...], 0.0)
    hfull_ref[...] = h
    hflat_ref[0] = h[:, :HALF]
    hflat_ref[1] = h[:, HALF:]


def _k4_call(h, agg, w1, b1, g1, bt1, w2, b2, g2, bt2, eps_arr):
    full = lambda shape: pl.BlockSpec(shape, lambda i: (0,) * len(shape))
    return pl.pallas_call(
        _k4_body,
        grid=(K4_NB,),
        in_specs=[
            pl.BlockSpec((K4_B, D), lambda i: (i, 0)),
            pl.BlockSpec((K4_B, HALF), lambda i: (i, 0)),
            pl.BlockSpec((K4_B, HALF), lambda i: (i + K4_NB, 0)),
            full((D, D)), full((D,)), full((D,)), full((D,)),
            full((D, D)), full((D,)), full((D,)), full((D,)),
            full((8, HALF)),
        ],
        out_specs=[
            pl.BlockSpec((K4_B, D), lambda i: (i, 0)),
            pl.BlockSpec((2, K4_B, HALF), lambda i: (0, i, 0)),
        ],
        out_shape=[
            jax.ShapeDtypeStruct((N_NODES, D), jnp.float32),
            jax.ShapeDtypeStruct((2, N_NODES, HALF), jnp.float32),
        ],
    )(h, agg, agg, w1, b1, g1, bt1, w2, b2, g2, bt2, eps_arr)


K5_B = 400
K5_NB = N_NODES // K5_B


def _k5_body(batch_ref, h1_ref, h2_ref, h3_ref, h4_ref, w1_ref, b1_ref,
             bg_ref, bb_ref, w2_ref, b2_ref, out_ref, pooled_sc, cnt_sc):
    i = pl.program_id(0)

    @pl.when(i == 0)
    def _():
        pooled_sc[...] = jnp.zeros_like(pooled_sc)
        cnt_sc[...] = jnp.zeros_like(cnt_sc)

    b = batch_ref[pl.ds(i, 1), 0, :]
    gids = lax.broadcasted_iota(jnp.int32, (N_GRAPHS, K5_B), 0)
    oh = (gids == b).astype(jnp.float32)
    hcat = jnp.concatenate(
        [h1_ref[...], h2_ref[...], h3_ref[...], h4_ref[...]], axis=1)
    pooled_sc[...] += jnp.dot(oh, hcat, preferred_element_type=jnp.float32)
    rs = jnp.sum(oh, axis=1, keepdims=True)
    cnt_sc[...] += jnp.broadcast_to(rs, (N_GRAPHS, HALF))

    @pl.when(i == K5_NB - 1)
    def _():
        cnt = jnp.maximum(cnt_sc[:, :1], 1.0)
        gm = pooled_sc[...] / cnt
        g = jnp.dot(gm, w1_ref[...], preferred_element_type=jnp.float32) + b1_ref[...]
        g = g * (BN_C * bg_ref[...]) + bb_ref[...]
        g = jnp.maximum(g, 0.0)
        logits = jnp.dot(g, w2_ref[...], preferred_element_type=jnp.float32) + b2_ref[...]
        col = lax.broadcasted_iota(jnp.int32, (N_GRAPHS, HALF), 1)
        masked = jnp.where(col < 16, logits, -3e38)
        m = jnp.max(masked, axis=1, keepdims=True)
        ex = jnp.where(col < 16, jnp.exp(logits - m), 0.0)
        lse = jnp.log(jnp.sum(ex, axis=1, keepdims=True))
        out_ref[...] = logits - m - lse


def _k5_call(batch3, h1, h2, h3, h4, w1, b1, bg, bb, w2p, b2p):
    full = lambda shape: pl.BlockSpec(shape, lambda i: (0,) * len(shape))
    hspec = pl.BlockSpec((K5_B, D), lambda i: (i, 0))
    return pl.pallas_call(
        _k5_body,
        grid=(K5_NB,),
        in_specs=[
            full((K5_NB, 1, K5_B)),
            hspec, hspec, hspec, hspec,
            full((NUM_LAYERS * D, D)), full((D,)),
            full((D,)), full((D,)),
            full((D, HALF)), full((HALF,)),
        ],
        out_specs=pl.BlockSpec((N_GRAPHS, HALF), lambda i: (0, 0)),
        out_shape=jax.ShapeDtypeStruct((N_GRAPHS, HALF), jnp.float32),
        scratch_shapes=[
            pltpu.VMEM((N_GRAPHS, NUM_LAYERS * D), jnp.float32),
            pltpu.VMEM((N_GRAPHS, HALF), jnp.float32),
        ],
    )(batch3, h1, h2, h3, h4, w1, b1, bg, bb, w2p, b2p)


def kernel(x, pos_enc, params, edge_index, batch, pos_index, pos_batch):
    src = edge_index[0]
    dst = edge_index[1]
    bounds = jnp.arange(0, N_EDGES + 1, EC, dtype=jnp.int32)
    pcnt = jnp.searchsorted(pos_batch, bounds).astype(jnp.int32)
    pcnt = jnp.pad(pcnt, (0, 816 - pcnt.shape[0]))

    zraw = _k1(params['z_init'], pos_index, pos_enc, pos_batch, pcnt)

    convs = [params['conv1']] + list(params['convs'])
    we = jnp.stack([cp['we'] for cp in convs])
    be = jnp.stack([cp['be'] for cp in convs])
    e = _k2_call(zraw, params['z_w'], params['z_b'], params['z_g1'],
                 params['z_bt1'], params['z_g2'], params['z_bt2'], we, be)
    eflat = e.reshape(2 * NUM_LAYERS * N_EDGES, HALF)

    h = x
    hflat = jnp.concatenate([x[:, :HALF], x[:, HALF:]], axis=0)
    hs = []
    for l, cp in enumerate(convs):
        larr = jnp.full((16,), l, jnp.int32)
        agg = _k3(hflat, eflat, src, dst, larr)
        eps_arr = jnp.full((8, HALF), 1.0 + cp['eps'], jnp.float32)
        h, hfl2 = _k4_call(h, agg, cp['w1'], cp['b1'], cp['g1'], cp['bt1'],
                           cp['w2'], cp['b2'], cp['g2'], cp['bt2'], eps_arr)
        hflat = hfl2.reshape(2 * N_NODES, HALF)
        hs.append(h)

    batch3 = batch.reshape(K5_NB, 1, K5_B)
    w2p = jnp.pad(params['lin2_w'], ((0, 0), (0, HALF - 16)))
    b2p = jnp.pad(params['lin2_b'], (0, HALF - 16))
    out = _k5_call(batch3, hs[0], hs[1], hs[2], hs[3],
                   params['lin1_w'], params['lin1_b'],
                   params['bn_g'], params['bn_b'], w2p, b2p)
    return out[:, :16]


# speedup vs baseline: 2.9398x; 1.3772x over previous
import functools

import jax
import jax.numpy as jnp
from jax import lax
from jax.experimental import pallas as pl
from jax.experimental.pallas import tpu as pltpu
from jax.experimental.pallas import tpu_sc as plsc

N_NODES = 10000
N_EDGES = 160000
D = 256
HALF = 128
NUM_LAYERS = 4
Z_IN = 1800
P_POS = 320000
N_GRAPHS = 128
BN_C = (1.0 + 1e-5) ** -0.5

def _sread(ref, i):
    return ref[pl.ds(i, 16)][0]


NW = 32
E_PER_W = N_EDGES // NW
EC = 200
PBUF = 128
K3_EW = N_EDGES // 16
K3_C = 40
K3_NC = K3_EW // K3_C
NODE_NC = N_NODES // K3_C


def _k1_body(zinit_hbm, pidx_hbm, pe_hbm, pb_hbm, pcnt_hbm, zraw_hbm,
             pcnt_v, pb_a, pe_a, pidx_a, pb_b, pe_b, pidx_b,
             zrows_a, zrows_b, acc_v,
             sem_ia, sem_ib, sem_ga, sem_gb, sem_f):
    c = lax.axis_index("c")
    s = lax.axis_index("s")
    w = s * 2 + c
    e0 = w * E_PER_W
    pltpu.async_copy(pcnt_hbm, pcnt_v, sem_ia).wait()
    n_chunks = E_PER_W // EC
    pmax = P_POS - PBUF

    def issue_idx(bb, pbv, pev, piv, sem):
        c1 = pltpu.async_copy(pb_hbm.at[pl.ds(bb, PBUF)], pbv.at[pl.ds(0, PBUF)], sem)
        c2 = pltpu.async_copy(pe_hbm.at[pl.ds(bb, PBUF)], pev.at[pl.ds(0, PBUF)], sem)
        c3 = pltpu.async_copy(pidx_hbm.at[pl.ds(bb, PBUF)], piv.at[pl.ds(0, PBUF)], sem)
        c1.wait()
        c2.wait()
        c3.wait()

    def issue_gather(piv, zrv, sem):
        pltpu.async_copy(zinit_hbm.at[piv.at[pl.ds(0, PBUF)]], zrv, sem)

    def wait_gather(piv, zrv, sem):
        pltpu.make_async_copy(zinit_hbm.at[piv.at[pl.ds(0, PBUF)]], zrv, sem).wait()

    def chunk_body(ci, _):
        base = e0 + ci * EC
        gi = w * n_chunks + ci
        p0c = _sread(pcnt_v, gi)
        p1c = _sread(pcnt_v, gi + 1)
        bb0 = (p0c // PBUF) * PBUF
        nb = jnp.where(p1c > p0c, (p1c - bb0 + PBUF - 1) // PBUF, 0)

        def bidx(k):
            return pl.multiple_of(jnp.minimum(bb0 + k * PBUF, pmax), PBUF)

        issue_idx(bidx(0), pb_a, pe_a, pidx_a, sem_ia)
        issue_gather(pidx_a, zrows_a, sem_ga)

        def zero_row(r, __):
            for k in range(D // 16):
                acc_v[r, pl.ds(16 * k, 16)] = jnp.zeros((16,), jnp.float32)
            return 0
        lax.fori_loop(0, EC + 8, zero_row, 0, unroll=False)

        def pos_loop(bb, be, pbv, pev, zrv):
            def pos(j, __):
                p = bb + j
                valid = (p >= p0c) & (p < p1c) & (be < nb)
                le = jnp.where(valid, _sread(pbv, j) - base, EC)
                pe = _sread(pev, j)
                vals = [zrv[j, pl.ds(16 * k, 16)] * pe for k in range(D // 16)]
                for k in range(D // 16):
                    plsc.addupdate(acc_v.at[le, pl.ds(16 * k, 16)], vals[k])
                return 0
            lax.fori_loop(0, PBUF, pos, 0, unroll=False)

        def pair(bp, __):
            e = 2 * bp
            issue_idx(bidx(e + 1), pb_b, pe_b, pidx_b, sem_ib)
            issue_gather(pidx_b, zrows_b, sem_gb)
            wait_gather(pidx_a, zrows_a, sem_ga)
            pos_loop(bidx(e), e, pb_a, pe_a, zrows_a)
            issue_idx(bidx(e + 2), pb_a, pe_a, pidx_a, sem_ia)
            issue_gather(pidx_a, zrows_a, sem_ga)
            wait_gather(pidx_b, zrows_b, sem_gb)
            pos_loop(bidx(e + 1), e + 1, pb_b, pe_b, zrows_b)
            return 0
        lax.fori_loop(0, (nb + 1) // 2, pair, 0, unroll=False)

        wait_gather(pidx_a, zrows_a, sem_ga)
        pltpu.async_copy(acc_v.at[pl.ds(0, EC)], zraw_hbm.at[pl.ds(base, EC)], sem_f).wait()
        return 0

    lax.fori_loop(0, n_chunks, chunk_body, 0, unroll=False)


_k1 = functools.partial(
    pl.kernel,
    out_type=jax.ShapeDtypeStruct((N_EDGES, D), jnp.float32),
    mesh=plsc.VectorSubcoreMesh(core_axis_name="c", subcore_axis_name="s", num_cores=2, num_subcores=16),
    scratch_types=[
        pltpu.VMEM((816,), jnp.int32),
        pltpu.VMEM((PBUF + 16,), jnp.int32),
        pltpu.VMEM((PBUF + 16,), jnp.float32),
        pltpu.VMEM((PBUF + 16,), jnp.int32),
        pltpu.VMEM((PBUF + 16,), jnp.int32),
        pltpu.VMEM((PBUF + 16,), jnp.float32),
        pltpu.VMEM((PBUF + 16,), jnp.int32),
        pltpu.VMEM((PBUF, D), jnp.float32),
        pltpu.VMEM((PBUF, D), jnp.float32),
        pltpu.VMEM((EC + 8, D), jnp.float32),
        pltpu.SemaphoreType.DMA,
        pltpu.SemaphoreType.DMA,
        pltpu.SemaphoreType.DMA,
        pltpu.SemaphoreType.DMA,
        pltpu.SemaphoreType.DMA,
    ],
)(_k1_body)


_IDX_OFFS = (0, 16, 24)


def _k3_body(hflat_hbm, eflat_hbm, src_hbm, dst_hbm, larr_hbm, agg_hbm,
             larr_v,
             sidx_a, didx_a, gidx_a, dsc_a, xbuf_a, ebuf_a, mbuf_a,
             sidx_b, didx_b, gidx_b, dsc_b, xbuf_b, ebuf_b, mbuf_b,
             acc_sp,
             sem_i, sem_ga, sem_gb, sem_ea, sem_eb, sem_sa, sem_sb, sem_w):
    c = lax.axis_index("c")
    s = lax.axis_index("s")
    pltpu.async_copy(larr_hbm, larr_v, sem_i).wait()
    lcbase = (larr_v[pl.ds(0, 16)][0] * 2 + c) * N_EDGES
    ew0 = s * K3_EW

    def zero_row(r, _):
        for k in range(HALF // 16):
            mbuf_a[r, pl.ds(16 * k, 16)] = jnp.zeros((16,), jnp.float32)
        return 0
    lax.fori_loop(0, K3_C, zero_row, 0, unroll=False)

    nloc = (NODE_NC - s + 15) // 16

    def zero_chunk(j, _):
        i = s + 16 * j
        pltpu.async_copy(mbuf_a, acc_sp.at[pl.ds(i * K3_C, K3_C)], sem_w).wait()
        return 0
    lax.fori_loop(0, nloc, zero_chunk, 0, unroll=False)
    plsc.subcore_barrier()

    def fetch(ci, sidx, didx, gidx, xbuf, ebuf, sem_g, sem_e):
        eb = pl.multiple_of(ew0 + ci * K3_C, 8)
        c1 = pltpu.async_copy(src_hbm.at[pl.ds(eb, K3_C)], sidx, sem_i)
        c2 = pltpu.async_copy(dst_hbm.at[pl.ds(eb, K3_C)], didx, sem_i)
        c1.wait()
        c2.wait()
        for o in _IDX_OFFS:
            sl = pl.ds(o, 16)
            gidx[sl] = sidx[sl] + c * N_NODES
        pltpu.async_copy(hflat_hbm.at[gidx], xbuf, sem_g)
        pltpu.async_copy(
            eflat_hbm.at[pl.ds(pl.multiple_of(lcbase + eb, 8), K3_C)], ebuf, sem_e)

    def wait_fetch(gidx, xbuf, ebuf, sem_g, sem_e):
        pltpu.make_async_copy(hflat_hbm.at[gidx], xbuf, sem_g).wait()
        pltpu.make_async_copy(eflat_hbm.at[pl.ds(0, K3_C)], ebuf, sem_e).wait()

    def wait_scatter(dsc, mbuf, sem):
        pltpu.make_async_copy(mbuf, acc_sp.at[dsc], sem).wait()

    def compute_issue(didx, dsc, xbuf, ebuf, mbuf, sem_s):
        def mrow(r, _):
            vals = [
                jnp.maximum(
                    xbuf[r, pl.ds(16 * k, 16)] + ebuf[r, pl.ds(16 * k, 16)], 0.0)
                for k in range(HALF // 16)
            ]
            for k in range(HALF // 16):
                mbuf[r, pl.ds(16 * k, 16)] = vals[k]
            return 0
        lax.fori_loop(0, K3_C, mrow, 0, unroll=False)
        for o in _IDX_OFFS:
            sl = pl.ds(o, 16)
            dsc[sl] = didx[sl]
        pltpu.async_copy(mbuf, acc_sp.at[dsc], sem_s, add=True)

    fetch(0, sidx_a, didx_a, gidx_a, xbuf_a, ebuf_a, sem_ga, sem_ea)
    fetch(1, sidx_b, didx_b, gidx_b, xbuf_b, ebuf_b, sem_gb, sem_eb)
    wait_fetch(gidx_a, xbuf_a, ebuf_a, sem_ga, sem_ea)
    compute_issue(didx_a, dsc_a, xbuf_a, ebuf_a, mbuf_a, sem_sa)
    fetch(2, sidx_a, didx_a, gidx_a, xbuf_a, ebuf_a, sem_ga, sem_ea)
    wait_fetch(gidx_b, xbuf_b, ebuf_b, sem_gb, sem_eb)
    compute_issue(didx_b, dsc_b, xbuf_b, ebuf_b, mbuf_b, sem_sb)
    fetch(3, sidx_b, didx_b, gidx_b, xbuf_b, ebuf_b, sem_gb, sem_eb)

    def pair(t, _):
        e = 2 * t
        wait_fetch(gidx_a, xbuf_a, ebuf_a, sem_ga, sem_ea)
        wait_scatter(dsc_a, mbuf_a, sem_sa)
        compute_issue(didx_a, dsc_a, xbuf_a, ebuf_a, mbuf_a, sem_sa)
        fetch(e + 2, sidx_a, didx_a, gidx_a, xbuf_a, ebuf_a, sem_ga, sem_ea)
        wait_fetch(gidx_b, xbuf_b, ebuf_b, sem_gb, sem_eb)
        wait_scatter(dsc_b, mbuf_b, sem_sb)
        compute_issue(didx_b, dsc_b, xbuf_b, ebuf_b, mbuf_b, sem_sb)
        fetch(e + 3, sidx_b, didx_b, gidx_b, xbuf_b, ebuf_b, sem_gb, sem_eb)
        return 0
    lax.fori_loop(1, K3_NC // 2 - 1, pair, 0, unroll=False)

    wait_fetch(gidx_a, xbuf_a, ebuf_a, sem_ga, sem_ea)
    wait_scatter(dsc_a, mbuf_a, sem_sa)
    compute_issue(didx_a, dsc_a, xbuf_a, ebuf_a, mbuf_a, sem_sa)
    wait_fetch(gidx_b, xbuf_b, ebuf_b, sem_gb, sem_eb)
    wait_scatter(dsc_b, mbuf_b, sem_sb)
    compute_issue(didx_b, dsc_b, xbuf_b, ebuf_b, mbuf_b, sem_sb)
    wait_scatter(dsc_a, mbuf_a, sem_sa)
    wait_scatter(dsc_b, mbuf_b, sem_sb)
    plsc.subcore_barrier()

    def wb_chunk(j, _):
        i = s + 16 * j
        pltpu.async_copy(acc_sp.at[pl.ds(i * K3_C, K3_C)], xbuf_a, sem_w).wait()
        pltpu.async_copy(xbuf_a, agg_hbm.at[pl.ds(pl.multiple_of(c * N_NODES + i * K3_C, 8), K3_C)], sem_w).wait()
        return 0
    lax.fori_loop(0, nloc, wb_chunk, 0, unroll=False)


_k3 = functools.partial(
    pl.kernel,
    out_type=jax.ShapeDtypeStruct((2 * N_NODES, HALF), jnp.float32),
    mesh=plsc.VectorSubcoreMesh(core_axis_name="c", subcore_axis_name="s", num_cores=2, num_subcores=16),
    scratch_types=[
        pltpu.VMEM((16,), jnp.int32),
        pltpu.VMEM((K3_C,), jnp.int32),
        pltpu.VMEM((K3_C,), jnp.int32),
        pltpu.VMEM((K3_C,), jnp.int32),
        pltpu.VMEM((K3_C,), jnp.int32),
        pltpu.VMEM((K3_C, HALF), jnp.float32),
        pltpu.VMEM((K3_C, HALF), jnp.float32),
        pltpu.VMEM((K3_C, HALF), jnp.float32),
        pltpu.VMEM((K3_C,), jnp.int32),
        pltpu.VMEM((K3_C,), jnp.int32),
        pltpu.VMEM((K3_C,), jnp.int32),
        pltpu.VMEM((K3_C,), jnp.int32),
        pltpu.VMEM((K3_C, HALF), jnp.float32),
        pltpu.VMEM((K3_C, HALF), jnp.float32),
        pltpu.VMEM((K3_C, HALF), jnp.float32),
        pltpu.VMEM_SHARED((N_NODES, HALF), jnp.float32),
        pltpu.SemaphoreType.DMA,
        pltpu.SemaphoreType.DMA,
        pltpu.SemaphoreType.DMA,
        pltpu.SemaphoreType.DMA,
        pltpu.SemaphoreType.DMA,
        pltpu.SemaphoreType.DMA,
        pltpu.SemaphoreType.DMA,
        pltpu.SemaphoreType.DMA,
    ],
)(_k3_body)


K2_B = 640
K2_NB = N_EDGES // K2_B


def _k2_body(zraw_ref, zw_ref, zb_ref, zg1_ref, zbt1_ref, zg2_ref, zbt2_ref,
             we_ref, be_ref, e_ref):
    z = zraw_ref[...]
    z1 = jnp.maximum(z * (BN_C * zg1_ref[...]) + zbt1_ref[...], 0.0)
    t = jnp.dot(z1, zw_ref[...], preferred_element_type=jnp.float32) + zb_ref[...]
    z2 = jnp.maximum(t * (BN_C * zg2_ref[...]) + zbt2_ref[...], 0.0)
    for l in range(NUM_LAYERS):
        el = jnp.dot(z2, we_ref[l], preferred_element_type=jnp.float32) + be_ref[l]
        e_ref[2 * l] = el[:, :HALF]
        e_ref[2 * l + 1] = el[:, HALF:]


def _k2_call(zraw, zw, zb, zg1, zbt1, zg2, zbt2, we, be):
    full = lambda shape: pl.BlockSpec(shape, lambda i: (0,) * len(shape))
    return pl.pallas_call(
        _k2_body,
        grid=(K2_NB,),
        in_specs=[
            pl.BlockSpec((K2_B, D), lambda i: (i, 0)),
            full((D, D)), full((D,)), full((D,)), full((D,)),
            full((D,)), full((D,)),
            full((NUM_LAYERS, D, D)), full((NUM_LAYERS, D)),
        ],
        out_specs=pl.BlockSpec((2 * NUM_LAYERS, K2_B, HALF), lambda i: (0, i, 0)),
        out_shape=jax.ShapeDtypeStruct((2 * NUM_LAYERS, N_EDGES, HALF), jnp.float32),
    )(zraw, zw, zb, zg1, zbt1, zg2, zbt2, we, be)


K4_B = 400
K4_NB = N_NODES // K4_B


def _k4_body(h_ref, aggl_ref, aggr_ref, w1_ref, b1_ref, g1_ref, bt1_ref,
             w2_ref, b2_ref, g2_ref, bt2_ref, eps_ref, hfull_ref, hflat_ref):
    agg = jnp.concatenate([aggl_ref[...], aggr_ref[...]], axis=1)
    hin = eps_ref[0, 0] * h_ref[...] + agg
    t1 = jnp.dot(hin, w1_ref[...], preferred_element_type=jnp.float32) + b1_ref[...]
    h1 = jnp.maximum(t1 * (BN_C * g1_ref[...]) + bt1_ref[...], 0.0)
    t2 = jnp.dot(h1, w2_ref[...], preferred_element_type=jnp.float32) + b2_ref[...]
    h = jnp.maximum(t2 * (BN_C * g2_ref[...]) + bt2_ref[...], 0.0)
    hfull_ref[...] = h
    hflat_ref[0] = h[:, :HALF]
    hflat_ref[1] = h[:, HALF:]


def _k4_call(h, agg, w1, b1, g1, bt1, w2, b2, g2, bt2, eps_arr):
    full = lambda shape: pl.BlockSpec(shape, lambda i: (0,) * len(shape))
    return pl.pallas_call(
        _k4_body,
        grid=(K4_NB,),
        in_specs=[
            pl.BlockSpec((K4_B, D), lambda i: (i, 0)),
            pl.BlockSpec((K4_B, HALF), lambda i: (i, 0)),
            pl.BlockSpec((K4_B, HALF), lambda i: (i + K4_NB, 0)),
            full((D, D)), full((D,)), full((D,)), full((D,)),
            full((D, D)), full((D,)), full((D,)), full((D,)),
            full((8, HALF)),
        ],
        out_specs=[
            pl.BlockSpec((K4_B, D), lambda i: (i, 0)),
            pl.BlockSpec((2, K4_B, HALF), lambda i: (0, i, 0)),
        ],
        out_shape=[
            jax.ShapeDtypeStruct((N_NODES, D), jnp.float32),
            jax.ShapeDtypeStruct((2, N_NODES, HALF), jnp.float32),
        ],
    )(h, agg, agg, w1, b1, g1, bt1, w2, b2, g2, bt2, eps_arr)


K5_B = 400
K5_NB = N_NODES // K5_B


def _k5_body(batch_ref, h1_ref, h2_ref, h3_ref, h4_ref, w1_ref, b1_ref,
             bg_ref, bb_ref, w2_ref, b2_ref, out_ref, pooled_sc, cnt_sc):
    i = pl.program_id(0)

    @pl.when(i == 0)
    def _():
        pooled_sc[...] = jnp.zeros_like(pooled_sc)
        cnt_sc[...] = jnp.zeros_like(cnt_sc)

    b = batch_ref[pl.ds(i, 1), 0, :]
    gids = lax.broadcasted_iota(jnp.int32, (N_GRAPHS, K5_B), 0)
    oh = (gids == b).astype(jnp.float32)
    hcat = jnp.concatenate(
        [h1_ref[...], h2_ref[...], h3_ref[...], h4_ref[...]], axis=1)
    pooled_sc[...] += jnp.dot(oh, hcat, preferred_element_type=jnp.float32)
    rs = jnp.sum(oh, axis=1, keepdims=True)
    cnt_sc[...] += jnp.broadcast_to(rs, (N_GRAPHS, HALF))

    @pl.when(i == K5_NB - 1)
    def _():
        cnt = jnp.maximum(cnt_sc[:, :1], 1.0)
        gm = pooled_sc[...] / cnt
        g = jnp.dot(gm, w1_ref[...], preferred_element_type=jnp.float32) + b1_ref[...]
        g = g * (BN_C * bg_ref[...]) + bb_ref[...]
        g = jnp.maximum(g, 0.0)
        logits = jnp.dot(g, w2_ref[...], preferred_element_type=jnp.float32) + b2_ref[...]
        col = lax.broadcasted_iota(jnp.int32, (N_GRAPHS, HALF), 1)
        masked = jnp.where(col < 16, logits, -3e38)
        m = jnp.max(masked, axis=1, keepdims=True)
        ex = jnp.where(col < 16, jnp.exp(logits - m), 0.0)
        lse = jnp.log(jnp.sum(ex, axis=1, keepdims=True))
        out_ref[...] = logits - m - lse


def _k5_call(batch3, h1, h2, h3, h4, w1, b1, bg, bb, w2p, b2p):
    full = lambda shape: pl.BlockSpec(shape, lambda i: (0,) * len(shape))
    hspec = pl.BlockSpec((K5_B, D), lambda i: (i, 0))
    return pl.pallas_call(
        _k5_body,
        grid=(K5_NB,),
        in_specs=[
            full((K5_NB, 1, K5_B)),
            hspec, hspec, hspec, hspec,
            full((NUM_LAYERS * D, D)), full((D,)),
            full((D,)), full((D,)),
            full((D, HALF)), full((HALF,)),
        ],
        out_specs=pl.BlockSpec((N_GRAPHS, HALF), lambda i: (0, 0)),
        out_shape=jax.ShapeDtypeStruct((N_GRAPHS, HALF), jnp.float32),
        scratch_shapes=[
            pltpu.VMEM((N_GRAPHS, NUM_LAYERS * D), jnp.float32),
            pltpu.VMEM((N_GRAPHS, HALF), jnp.float32),
        ],
    )(batch3, h1, h2, h3, h4, w1, b1, bg, bb, w2p, b2p)


def kernel(x, pos_enc, params, edge_index, batch, pos_index, pos_batch):
    src = edge_index[0]
    dst = edge_index[1]
    bounds = jnp.arange(0, N_EDGES + 1, EC, dtype=jnp.int32)
    pcnt = jnp.searchsorted(pos_batch, bounds).astype(jnp.int32)
    pcnt = jnp.pad(pcnt, (0, 816 - pcnt.shape[0]))

    zraw = _k1(params['z_init'], pos_index, pos_enc, pos_batch, pcnt)

    convs = [params['conv1']] + list(params['convs'])
    we = jnp.stack([cp['we'] for cp in convs])
    be = jnp.stack([cp['be'] for cp in convs])
    e = _k2_call(zraw, params['z_w'], params['z_b'], params['z_g1'],
                 params['z_bt1'], params['z_g2'], params['z_bt2'], we, be)
    eflat = e.reshape(2 * NUM_LAYERS * N_EDGES, HALF)

    h = x
    hflat = jnp.concatenate([x[:, :HALF], x[:, HALF:]], axis=0)
    hs = []
    for l, cp in enumerate(convs):
        larr = jnp.full((16,), l, jnp.int32)
        agg = _k3(hflat, eflat, src, dst, larr)
        eps_arr = jnp.full((8, HALF), 1.0 + cp['eps'], jnp.float32)
        h, hfl2 = _k4_call(h, agg, cp['w1'], cp['b1'], cp['g1'], cp['bt1'],
                           cp['w2'], cp['b2'], cp['g2'], cp['bt2'], eps_arr)
        hflat = hfl2.reshape(2 * N_NODES, HALF)
        hs.append(h)

    batch3 = batch.reshape(K5_NB, 1, K5_B)
    w2p = jnp.pad(params['lin2_w'], ((0, 0), (0, HALF - 16)))
    b2p = jnp.pad(params['lin2_b'], (0, HALF - 16))
    out = _k5_call(batch3, hs[0], hs[1], hs[2], hs[3],
                   params['lin1_w'], params['lin1_b'],
                   params['bn_g'], params['bn_b'], w2p, b2p)
    return out[:, :16]


# speedup vs baseline: 3.0528x; 1.0384x over previous
import functools

import jax
import jax.numpy as jnp
from jax import lax
from jax.experimental import pallas as pl
from jax.experimental.pallas import tpu as pltpu
from jax.experimental.pallas import tpu_sc as plsc

N_NODES = 10000
N_EDGES = 160000
D = 256
HALF = 128
NUM_LAYERS = 4
Z_IN = 1800
P_POS = 320000
N_GRAPHS = 128
BN_C = (1.0 + 1e-5) ** -0.5

def _sread(ref, i):
    return ref[pl.ds(i, 16)][0]


NW = 32
E_PER_W = N_EDGES // NW
EC = 200
PBUF = 128
K3_EW = N_EDGES // 16
K3_C = 40
K3_NC = K3_EW // K3_C
NODE_NC = N_NODES // K3_C


def _k1_body(zinit_hbm, pidx_hbm, pe_hbm, pb_hbm, pcnt_hbm, zraw_hbm,
             pcnt_v, pb_a, pe_a, pidx_a, pb_b, pe_b, pidx_b,
             zrows_a, zrows_b, acc_v,
             sem_ia, sem_ib, sem_ga, sem_gb, sem_f):
    c = lax.axis_index("c")
    s = lax.axis_index("s")
    w = s * 2 + c
    e0 = w * E_PER_W
    pltpu.async_copy(pcnt_hbm, pcnt_v, sem_ia).wait()
    n_chunks = E_PER_W // EC
    pmax = P_POS - PBUF

    def issue_idx(bb, pbv, pev, piv, sem):
        c1 = pltpu.async_copy(pb_hbm.at[pl.ds(bb, PBUF)], pbv.at[pl.ds(0, PBUF)], sem)
        c2 = pltpu.async_copy(pe_hbm.at[pl.ds(bb, PBUF)], pev.at[pl.ds(0, PBUF)], sem)
        c3 = pltpu.async_copy(pidx_hbm.at[pl.ds(bb, PBUF)], piv.at[pl.ds(0, PBUF)], sem)
        c1.wait()
        c2.wait()
        c3.wait()

    def issue_gather(piv, zrv, sem):
        pltpu.async_copy(zinit_hbm.at[piv.at[pl.ds(0, PBUF)]], zrv, sem)

    def wait_gather(piv, zrv, sem):
        pltpu.make_async_copy(zinit_hbm.at[piv.at[pl.ds(0, PBUF)]], zrv, sem).wait()

    def chunk_body(ci, _):
        base = e0 + ci * EC
        gi = w * n_chunks + ci
        p0c = _sread(pcnt_v, gi)
        p1c = _sread(pcnt_v, gi + 1)
        bb0 = (p0c // PBUF) * PBUF
        nb = jnp.where(p1c > p0c, (p1c - bb0 + PBUF - 1) // PBUF, 0)

        def bidx(k):
            return pl.multiple_of(jnp.minimum(bb0 + k * PBUF, pmax), PBUF)

        issue_idx(bidx(0), pb_a, pe_a, pidx_a, sem_ia)
        issue_gather(pidx_a, zrows_a, sem_ga)

        def zero_row(r, __):
            for k in range(D // 16):
                acc_v[r, pl.ds(16 * k, 16)] = jnp.zeros((16,), jnp.float32)
            return 0
        lax.fori_loop(0, EC + 8, zero_row, 0, unroll=False)

        def pos_loop(bb, be, pbv, pev, zrv):
            def one(j):
                p = bb + j
                valid = (p >= p0c) & (p < p1c) & (be < nb)
                le = jnp.where(valid, _sread(pbv, j) - base, EC)
                pe = _sread(pev, j)
                vals = [zrv[j, pl.ds(16 * k, 16)] * pe for k in range(D // 16)]
                return le, vals

            def pos(i, __):
                j = 2 * i
                le0, vals0 = one(j)
                le1, vals1 = one(j + 1)
                for k in range(D // 16):
                    plsc.addupdate(acc_v.at[le0, pl.ds(16 * k, 16)], vals0[k])
                for k in range(D // 16):
                    plsc.addupdate(acc_v.at[le1, pl.ds(16 * k, 16)], vals1[k])
                return 0
            lax.fori_loop(0, PBUF // 2, pos, 0, unroll=False)

        def pair(bp, __):
            e = 2 * bp
            issue_idx(bidx(e + 1), pb_b, pe_b, pidx_b, sem_ib)
            issue_gather(pidx_b, zrows_b, sem_gb)
            wait_gather(pidx_a, zrows_a, sem_ga)
            pos_loop(bidx(e), e, pb_a, pe_a, zrows_a)
            issue_idx(bidx(e + 2), pb_a, pe_a, pidx_a, sem_ia)
            issue_gather(pidx_a, zrows_a, sem_ga)
            wait_gather(pidx_b, zrows_b, sem_gb)
            pos_loop(bidx(e + 1), e + 1, pb_b, pe_b, zrows_b)
            return 0
        lax.fori_loop(0, (nb + 1) // 2, pair, 0, unroll=False)

        wait_gather(pidx_a, zrows_a, sem_ga)
        pltpu.async_copy(acc_v.at[pl.ds(0, EC)], zraw_hbm.at[pl.ds(base, EC)], sem_f).wait()
        return 0

    lax.fori_loop(0, n_chunks, chunk_body, 0, unroll=False)


_k1 = functools.partial(
    pl.kernel,
    out_type=jax.ShapeDtypeStruct((N_EDGES, D), jnp.float32),
    mesh=plsc.VectorSubcoreMesh(core_axis_name="c", subcore_axis_name="s", num_cores=2, num_subcores=16),
    scratch_types=[
        pltpu.VMEM((816,), jnp.int32),
        pltpu.VMEM((PBUF + 16,), jnp.int32),
        pltpu.VMEM((PBUF + 16,), jnp.float32),
        pltpu.VMEM((PBUF + 16,), jnp.int32),
        pltpu.VMEM((PBUF + 16,), jnp.int32),
        pltpu.VMEM((PBUF + 16,), jnp.float32),
        pltpu.VMEM((PBUF + 16,), jnp.int32),
        pltpu.VMEM((PBUF, D), jnp.float32),
        pltpu.VMEM((PBUF, D), jnp.float32),
        pltpu.VMEM((EC + 8, D), jnp.float32),
        pltpu.SemaphoreType.DMA,
        pltpu.SemaphoreType.DMA,
        pltpu.SemaphoreType.DMA,
        pltpu.SemaphoreType.DMA,
        pltpu.SemaphoreType.DMA,
    ],
)(_k1_body)


_IDX_OFFS = (0, 16, 24)


def _k3_body(hflat_hbm, eflat_hbm, src_hbm, dst_hbm, larr_hbm, agg_hbm,
             larr_v,
             sidx_a, didx_a, gidx_a, dsc_a, xbuf_a, ebuf_a, mbuf_a,
             sidx_b, didx_b, gidx_b, dsc_b, xbuf_b, ebuf_b, mbuf_b,
             acc_sp,
             sem_i, sem_ga, sem_gb, sem_ea, sem_eb, sem_sa, sem_sb, sem_w):
    c = lax.axis_index("c")
    s = lax.axis_index("s")
    pltpu.async_copy(larr_hbm, larr_v, sem_i).wait()
    lcbase = (larr_v[pl.ds(0, 16)][0] * 2 + c) * N_EDGES
    ew0 = s * K3_EW

    def zero_row(r, _):
        for k in range(HALF // 16):
            mbuf_a[r, pl.ds(16 * k, 16)] = jnp.zeros((16,), jnp.float32)
        return 0
    lax.fori_loop(0, K3_C, zero_row, 0, unroll=False)

    nloc = (NODE_NC - s + 15) // 16

    def zero_chunk(j, _):
        i = s + 16 * j
        pltpu.async_copy(mbuf_a, acc_sp.at[pl.ds(i * K3_C, K3_C)], sem_w).wait()
        return 0
    lax.fori_loop(0, nloc, zero_chunk, 0, unroll=False)
    plsc.subcore_barrier()

    def fetch(ci, sidx, didx, gidx, xbuf, ebuf, sem_g, sem_e):
        eb = pl.multiple_of(ew0 + ci * K3_C, 8)
        c1 = pltpu.async_copy(src_hbm.at[pl.ds(eb, K3_C)], sidx, sem_i)
        c2 = pltpu.async_copy(dst_hbm.at[pl.ds(eb, K3_C)], didx, sem_i)
        c1.wait()
        c2.wait()
        for o in _IDX_OFFS:
            sl = pl.ds(o, 16)
            gidx[sl] = sidx[sl] + c * N_NODES
        pltpu.async_copy(hflat_hbm.at[gidx], xbuf, sem_g)
        pltpu.async_copy(
            eflat_hbm.at[pl.ds(pl.multiple_of(lcbase + eb, 8), K3_C)], ebuf, sem_e)

    def wait_fetch(gidx, xbuf, ebuf, sem_g, sem_e):
        pltpu.make_async_copy(hflat_hbm.at[gidx], xbuf, sem_g).wait()
        pltpu.make_async_copy(eflat_hbm.at[pl.ds(0, K3_C)], ebuf, sem_e).wait()

    def wait_scatter(dsc, mbuf, sem):
        pltpu.make_async_copy(mbuf, acc_sp.at[dsc], sem).wait()

    def compute_issue(didx, dsc, xbuf, ebuf, mbuf, sem_s):
        def mrow(r, _):
            vals = [
                jnp.maximum(
                    xbuf[r, pl.ds(16 * k, 16)] + ebuf[r, pl.ds(16 * k, 16)], 0.0)
                for k in range(HALF // 16)
            ]
            for k in range(HALF // 16):
                mbuf[r, pl.ds(16 * k, 16)] = vals[k]
            return 0
        lax.fori_loop(0, K3_C, mrow, 0, unroll=False)
        for o in _IDX_OFFS:
            sl = pl.ds(o, 16)
            dsc[sl] = didx[sl]
        pltpu.async_copy(mbuf, acc_sp.at[dsc], sem_s, add=True)

    fetch(0, sidx_a, didx_a, gidx_a, xbuf_a, ebuf_a, sem_ga, sem_ea)
    fetch(1, sidx_b, didx_b, gidx_b, xbuf_b, ebuf_b, sem_gb, sem_eb)
    wait_fetch(gidx_a, xbuf_a, ebuf_a, sem_ga, sem_ea)
    compute_issue(didx_a, dsc_a, xbuf_a, ebuf_a, mbuf_a, sem_sa)
    fetch(2, sidx_a, didx_a, gidx_a, xbuf_a, ebuf_a, sem_ga, sem_ea)
    wait_fetch(gidx_b, xbuf_b, ebuf_b, sem_gb, sem_eb)
    compute_issue(didx_b, dsc_b, xbuf_b, ebuf_b, mbuf_b, sem_sb)
    fetch(3, sidx_b, didx_b, gidx_b, xbuf_b, ebuf_b, sem_gb, sem_eb)

    def pair(t, _):
        e = 2 * t
        wait_fetch(gidx_a, xbuf_a, ebuf_a, sem_ga, sem_ea)
        wait_scatter(dsc_a, mbuf_a, sem_sa)
        compute_issue(didx_a, dsc_a, xbuf_a, ebuf_a, mbuf_a, sem_sa)
        fetch(e + 2, sidx_a, didx_a, gidx_a, xbuf_a, ebuf_a, sem_ga, sem_ea)
        wait_fetch(gidx_b, xbuf_b, ebuf_b, sem_gb, sem_eb)
        wait_scatter(dsc_b, mbuf_b, sem_sb)
        compute_issue(didx_b, dsc_b, xbuf_b, ebuf_b, mbuf_b, sem_sb)
        fetch(e + 3, sidx_b, didx_b, gidx_b, xbuf_b, ebuf_b, sem_gb, sem_eb)
        return 0
    lax.fori_loop(1, K3_NC // 2 - 1, pair, 0, unroll=False)

    wait_fetch(gidx_a, xbuf_a, ebuf_a, sem_ga, sem_ea)
    wait_scatter(dsc_a, mbuf_a, sem_sa)
    compute_issue(didx_a, dsc_a, xbuf_a, ebuf_a, mbuf_a, sem_sa)
    wait_fetch(gidx_b, xbuf_b, ebuf_b, sem_gb, sem_eb)
    wait_scatter(dsc_b, mbuf_b, sem_sb)
    compute_issue(didx_b, dsc_b, xbuf_b, ebuf_b, mbuf_b, sem_sb)
    wait_scatter(dsc_a, mbuf_a, sem_sa)
    wait_scatter(dsc_b, mbuf_b, sem_sb)
    plsc.subcore_barrier()

    def wb_chunk(j, _):
        i = s + 16 * j
        pltpu.async_copy(acc_sp.at[pl.ds(i * K3_C, K3_C)], xbuf_a, sem_w).wait()
        pltpu.async_copy(xbuf_a, agg_hbm.at[pl.ds(pl.multiple_of(c * N_NODES + i * K3_C, 8), K3_C)], sem_w).wait()
        return 0
    lax.fori_loop(0, nloc, wb_chunk, 0, unroll=False)


_k3 = functools.partial(
    pl.kernel,
    out_type=jax.ShapeDtypeStruct((2 * N_NODES, HALF), jnp.float32),
    mesh=plsc.VectorSubcoreMesh(core_axis_name="c", subcore_axis_name="s", num_cores=2, num_subcores=16),
    scratch_types=[
        pltpu.VMEM((16,), jnp.int32),
        pltpu.VMEM((K3_C,), jnp.int32),
        pltpu.VMEM((K3_C,), jnp.int32),
        pltpu.VMEM((K3_C,), jnp.int32),
        pltpu.VMEM((K3_C,), jnp.int32),
        pltpu.VMEM((K3_C, HALF), jnp.float32),
        pltpu.VMEM((K3_C, HALF), jnp.float32),
        pltpu.VMEM((K3_C, HALF), jnp.float32),
        pltpu.VMEM((K3_C,), jnp.int32),
        pltpu.VMEM((K3_C,), jnp.int32),
        pltpu.VMEM((K3_C,), jnp.int32),
        pltpu.VMEM((K3_C,), jnp.int32),
        pltpu.VMEM((K3_C, HALF), jnp.float32),
        pltpu.VMEM((K3_C, HALF), jnp.float32),
        pltpu.VMEM((K3_C, HALF), jnp.float32),
        pltpu.VMEM_SHARED((N_NODES, HALF), jnp.float32),
        pltpu.SemaphoreType.DMA,
        pltpu.SemaphoreType.DMA,
        pltpu.SemaphoreType.DMA,
        pltpu.SemaphoreType.DMA,
        pltpu.SemaphoreType.DMA,
        pltpu.SemaphoreType.DMA,
        pltpu.SemaphoreType.DMA,
        pltpu.SemaphoreType.DMA,
    ],
)(_k3_body)


K2_B = 640
K2_NB = N_EDGES // K2_B


def _k2_body(zraw_ref, zw_ref, zb_ref, zg1_ref, zbt1_ref, zg2_ref, zbt2_ref,
             we_ref, be_ref, e_ref):
    z = zraw_ref[...]
    z1 = jnp.maximum(z * (BN_C * zg1_ref[...]) + zbt1_ref[...], 0.0)
    t = jnp.dot(z1, zw_ref[...], preferred_element_type=jnp.float32) + zb_ref[...]
    z2 = jnp.maximum(t * (BN_C * zg2_ref[...]) + zbt2_ref[...], 0.0)
    for l in range(NUM_LAYERS):
        el = jnp.dot(z2, we_ref[l], preferred_element_type=jnp.float32) + be_ref[l]
        e_ref[2 * l] = el[:, :HALF]
        e_ref[2 * l + 1] = el[:, HALF:]


def _k2_call(zraw, zw, zb, zg1, zbt1, zg2, zbt2, we, be):
    full = lambda shape: pl.BlockSpec(shape, lambda i: (0,) * len(shape))
    return pl.pallas_call(
        _k2_body,
        grid=(K2_NB,),
        in_specs=[
            pl.BlockSpec((K2_B, D), lambda i: (i, 0)),
            full((D, D)), full((D,)), full((D,)), full((D,)),
            full((D,)), full((D,)),
            full((NUM_LAYERS, D, D)), full((NUM_LAYERS, D)),
        ],
        out_specs=pl.BlockSpec((2 * NUM_LAYERS, K2_B, HALF), lambda i: (0, i, 0)),
        out_shape=jax.ShapeDtypeStruct((2 * NUM_LAYERS, N_EDGES, HALF), jnp.float32),
    )(zraw, zw, zb, zg1, zbt1, zg2, zbt2, we, be)


K4_B = 400
K4_NB = N_NODES // K4_B


def _k4_body(h_ref, aggl_ref, aggr_ref, w1_ref, b1_ref, g1_ref, bt1_ref,
             w2_ref, b2_ref, g2_ref, bt2_ref, eps_ref, hfull_ref, hflat_ref):
    agg = jnp.concatenate([aggl_ref[...], aggr_ref[...]], axis=1)
    hin = eps_ref[0, 0] * h_ref[...] + agg
    t1 = jnp.dot(hin, w1_ref[...], preferred_element_type=jnp.float32) + b1_ref[...]
    h1 = jnp.maximum(t1 * (BN_C * g1_ref[...]) + bt1_ref[...], 0.0)
    t2 = jnp.dot(h1, w2_ref[...], preferred_element_type=jnp.float32) + b2_ref[...]
    h = jnp.maximum(t2 * (BN_C * g2_ref[...]) + bt2_ref[...], 0.0)
    hfull_ref[...] = h
    hflat_ref[0] = h[:, :HALF]
    hflat_ref[1] = h[:, HALF:]


def _k4_call(h, agg, w1, b1, g1, bt1, w2, b2, g2, bt2, eps_arr):
    full = lambda shape: pl.BlockSpec(shape, lambda i: (0,) * len(shape))
    return pl.pallas_call(
        _k4_body,
        grid=(K4_NB,),
        in_specs=[
            pl.BlockSpec((K4_B, D), lambda i: (i, 0)),
            pl.BlockSpec((K4_B, HALF), lambda i: (i, 0)),
            pl.BlockSpec((K4_B, HALF), lambda i: (i + K4_NB, 0)),
            full((D, D)), full((D,)), full((D,)), full((D,)),
            full((D, D)), full((D,)), full((D,)), full((D,)),
            full((8, HALF)),
        ],
        out_specs=[
            pl.BlockSpec((K4_B, D), lambda i: (i, 0)),
            pl.BlockSpec((2, K4_B, HALF), lambda i: (0, i, 0)),
        ],
        out_shape=[
            jax.ShapeDtypeStruct((N_NODES, D), jnp.float32),
            jax.ShapeDtypeStruct((2, N_NODES, HALF), jnp.float32),
        ],
    )(h, agg, agg, w1, b1, g1, bt1, w2, b2, g2, bt2, eps_arr)


K5_B = 400
K5_NB = N_NODES // K5_B


def _k5_body(batch_ref, h1_ref, h2_ref, h3_ref, h4_ref, w1_ref, b1_ref,
             bg_ref, bb_ref, w2_ref, b2_ref, out_ref, pooled_sc, cnt_sc):
    i = pl.program_id(0)

    @pl.when(i == 0)
    def _():
        pooled_sc[...] = jnp.zeros_like(pooled_sc)
        cnt_sc[...] = jnp.zeros_like(cnt_sc)

    b = batch_ref[pl.ds(i, 1), 0, :]
    gids = lax.broadcasted_iota(jnp.int32, (N_GRAPHS, K5_B), 0)
    oh = (gids == b).astype(jnp.float32)
    hcat = jnp.concatenate(
        [h1_ref[...], h2_ref[...], h3_ref[...], h4_ref[...]], axis=1)
    pooled_sc[...] += jnp.dot(oh, hcat, preferred_element_type=jnp.float32)
    rs = jnp.sum(oh, axis=1, keepdims=True)
    cnt_sc[...] += jnp.broadcast_to(rs, (N_GRAPHS, HALF))

    @pl.when(i == K5_NB - 1)
    def _():
        cnt = jnp.maximum(cnt_sc[:, :1], 1.0)
        gm = pooled_sc[...] / cnt
        g = jnp.dot(gm, w1_ref[...], preferred_element_type=jnp.float32) + b1_ref[...]
        g = g * (BN_C * bg_ref[...]) + bb_ref[...]
        g = jnp.maximum(g, 0.0)
        logits = jnp.dot(g, w2_ref[...], preferred_element_type=jnp.float32) + b2_ref[...]
        col = lax.broadcasted_iota(jnp.int32, (N_GRAPHS, HALF), 1)
        masked = jnp.where(col < 16, logits, -3e38)
        m = jnp.max(masked, axis=1, keepdims=True)
        ex = jnp.where(col < 16, jnp.exp(logits - m), 0.0)
        lse = jnp.log(jnp.sum(ex, axis=1, keepdims=True))
        out_ref[...] = logits - m - lse


def _k5_call(batch3, h1, h2, h3, h4, w1, b1, bg, bb, w2p, b2p):
    full = lambda shape: pl.BlockSpec(shape, lambda i: (0,) * len(shape))
    hspec = pl.BlockSpec((K5_B, D), lambda i: (i, 0))
    return pl.pallas_call(
        _k5_body,
        grid=(K5_NB,),
        in_specs=[
            full((K5_NB, 1, K5_B)),
            hspec, hspec, hspec, hspec,
            full((NUM_LAYERS * D, D)), full((D,)),
            full((D,)), full((D,)),
            full((D, HALF)), full((HALF,)),
        ],
        out_specs=pl.BlockSpec((N_GRAPHS, HALF), lambda i: (0, 0)),
        out_shape=jax.ShapeDtypeStruct((N_GRAPHS, HALF), jnp.float32),
        scratch_shapes=[
            pltpu.VMEM((N_GRAPHS, NUM_LAYERS * D), jnp.float32),
            pltpu.VMEM((N_GRAPHS, HALF), jnp.float32),
        ],
    )(batch3, h1, h2, h3, h4, w1, b1, bg, bb, w2p, b2p)


def kernel(x, pos_enc, params, edge_index, batch, pos_index, pos_batch):
    src = edge_index[0]
    dst = edge_index[1]
    bounds = jnp.arange(0, N_EDGES + 1, EC, dtype=jnp.int32)
    pcnt = jnp.searchsorted(pos_batch, bounds).astype(jnp.int32)
    pcnt = jnp.pad(pcnt, (0, 816 - pcnt.shape[0]))

    zraw = _k1(params['z_init'], pos_index, pos_enc, pos_batch, pcnt)

    convs = [params['conv1']] + list(params['convs'])
    we = jnp.stack([cp['we'] for cp in convs])
    be = jnp.stack([cp['be'] for cp in convs])
    e = _k2_call(zraw, params['z_w'], params['z_b'], params['z_g1'],
                 params['z_bt1'], params['z_g2'], params['z_bt2'], we, be)
    eflat = e.reshape(2 * NUM_LAYERS * N_EDGES, HALF)

    h = x
    hflat = jnp.concatenate([x[:, :HALF], x[:, HALF:]], axis=0)
    hs = []
    for l, cp in enumerate(convs):
        larr = jnp.full((16,), l, jnp.int32)
        agg = _k3(hflat, eflat, src, dst, larr)
        eps_arr = jnp.full((8, HALF), 1.0 + cp['eps'], jnp.float32)
        h, hfl2 = _k4_call(h, agg, cp['w1'], cp['b1'], cp['g1'], cp['bt1'],
                           cp['w2'], cp['b2'], cp['g2'], cp['bt2'], eps_arr)
        hflat = hfl2.reshape(2 * N_NODES, HALF)
        hs.append(h)

    batch3 = batch.reshape(K5_NB, 1, K5_B)
    w2p = jnp.pad(params['lin2_w'], ((0, 0), (0, HALF - 16)))
    b2p = jnp.pad(params['lin2_b'], (0, HALF - 16))
    out = _k5_call(batch3, hs[0], hs[1], hs[2], hs[3],
                   params['lin1_w'], params['lin1_b'],
                   params['bn_g'], params['bn_b'], w2p, b2p)
    return out[:, :16]
